# baseline (speedup 1.0000x reference)
import dataclasses
import functools

import jax
import jax.numpy as jnp
import numpy as np
from jax import lax
from jax.experimental import pallas as pl
from jax.experimental.pallas import tpu as pltpu
from jax.experimental.pallas import tpu_sc as plsc

MAX_DEPTH = 4
NF = 8
B, T, H, TL = 32, 64, 512, 264
N = B * T
KPAD = 384
NP = 3072
NBLK = NP // 128
NW = 32
CHUNK = N // NW

@functools.cache
def _mesh():
    return plsc.VectorSubcoreMesh(core_axis_name="c", subcore_axis_name="s")


def _sc_params():
    cp = pltpu.CompilerParams()
    if "needs_layout_passes" in pltpu.CompilerParams.__dataclass_fields__:
        cp = dataclasses.replace(cp, needs_layout_passes=False)
    return cp


def _leaky(x):
    return jnp.where(x >= 0, x, 0.2 * x)


def _mm(a, b):
    return a @ b


def _wid():
    return lax.axis_index("s") * 2 + lax.axis_index("c")


def _s1a_body(gt8_hbm, cmd_hbm, hist_hbm, gt8_v, cmd_v, hist_v):
    wid = _wid()
    base = wid * CHUNK
    pltpu.sync_copy(gt8_hbm.at[pl.ds(base, CHUNK)], gt8_v)
    lane = lax.iota(jnp.int32, 16)
    hist = jnp.zeros((16,), jnp.int32)
    for g in range(CHUNK // 16):
        rows = lane + g * 16
        best = plsc.load_gather(gt8_v, [rows, jnp.zeros((16,), jnp.int32)])
        bi = jnp.zeros((16,), jnp.int32)
        for j in range(1, NF):
            colj = plsc.load_gather(
                gt8_v, [rows, jnp.full((16,), j, jnp.int32)])
            m = colj > best
            bi = jnp.where(m, j, bi)
            best = jnp.where(m, colj, best)
        cmd_v[pl.ds(g * 16, 16)] = bi
        for e in range(NF):
            cnt = plsc.all_reduce_population_count(bi == e)
            hist = hist + jnp.where(lane == e, cnt, 0)
    hist_v[...] = hist
    pltpu.sync_copy(cmd_v, cmd_hbm.at[pl.ds(base, CHUNK)])
    pltpu.sync_copy(hist_v, hist_hbm.at[wid])


def _s1a_call(gt8):
    k = pl.kernel(
        _s1a_body,
        out_type=[jax.ShapeDtypeStruct((N,), jnp.int32),
                  jax.ShapeDtypeStruct((NW, 16), jnp.int32)],
        mesh=_mesh(),
        scratch_types=[pltpu.VMEM((CHUNK, NF), jnp.float32),
                       pltpu.VMEM((CHUNK,), jnp.int32),
                       pltpu.VMEM((16,), jnp.int32)],
        compiler_params=_sc_params(),
    )
    return k(gt8)


def _s1b_body(cmd_hbm, hist_hbm, gtbb_hbm,
              pos_hbm, xgtbb_hbm, blk_hbm,
              cmd_v, hist_v, pos_v, gtbb_v, blk_v):
    wid = _wid()
    base = wid * CHUNK
    lane = lax.iota(jnp.int32, 16)
    pltpu.sync_copy(cmd_hbm.at[pl.ds(base, CHUNK)], cmd_v)
    pltpu.sync_copy(hist_hbm, hist_v)
    pltpu.sync_copy(gtbb_hbm.at[pl.ds(base, CHUNK)], gtbb_v)

    tot = jnp.zeros((16,), jnp.int32)
    prefix = jnp.zeros((16,), jnp.int32)
    for w in range(NW):
        row = hist_v[w]
        tot = tot + row
        prefix = prefix + row * ((w < wid).astype(jnp.int32))
    pc = ((tot + 127) >> 7) << 7
    po = jnp.cumsum(pc) - pc
    basev = po + prefix

    for g in range(CHUNK // 16):
        cm = cmd_v[pl.ds(g * 16, 16)]
        dest = jnp.zeros((16,), jnp.int32)
        for e in range(NF):
            m = cm == e
            rk = jnp.cumsum(m.astype(jnp.int32)) - 1
            be = jnp.sum(jnp.where(lane == e, basev, 0))
            dest = jnp.where(m, be + rk, dest)
            cnt = plsc.all_reduce_population_count(m)
            basev = basev + jnp.where(lane == e, cnt, 0)
        pos_v[0, pl.ds(g * 16, 16)] = dest

    pltpu.sync_copy(gtbb_v, xgtbb_hbm.at[pos_v.at[0]])
    pltpu.sync_copy(pos_v, pos_hbm.at[wid])

    @pl.when(wid == 0)
    def _():
        blo = jnp.zeros((16,), jnp.int32)
        bhi = jnp.zeros((16,), jnp.int32)
        k1 = lane * 128
        k2 = (lane + 16) * 128
        for e in range(NF):
            po_e = jnp.sum(jnp.where(lane == e, po, 0))
            pc_e = jnp.sum(jnp.where(lane == e, pc, 0))
            blo = jnp.where((k1 >= po_e) & (k1 < po_e + pc_e), e, blo)
            bhi = jnp.where((k2 >= po_e) & (k2 < po_e + pc_e), e, bhi)
        blk_v[pl.ds(0, 16)] = blo
        blk_v[pl.ds(16, 16)] = bhi
        pltpu.sync_copy(blk_v, blk_hbm)


def _s1b_call(cmd, hist, gtbb):
    k = pl.kernel(
        _s1b_body,
        out_type=[jax.ShapeDtypeStruct((NW, 1, CHUNK), jnp.int32),
                  jax.ShapeDtypeStruct((NP, KPAD), jnp.float32),
                  jax.ShapeDtypeStruct((NW,), jnp.int32)],
        mesh=_mesh(),
        scratch_types=[pltpu.VMEM((CHUNK,), jnp.int32),
                       pltpu.VMEM((NW, 16), jnp.int32),
                       pltpu.VMEM((1, CHUNK), jnp.int32),
                       pltpu.VMEM((CHUNK, KPAD), jnp.float32),
                       pltpu.VMEM((NW,), jnp.int32)],
        compiler_params=_sc_params(),
    )
    return k(cmd, hist, gtbb)


def _s1c_body(g_hbm, pos_hbm, xg_hbm, pos_v, g_v):
    wid = _wid()
    pltpu.sync_copy(pos_hbm.at[wid], pos_v)
    pltpu.sync_copy(g_hbm.at[pl.ds(wid * CHUNK, CHUNK)], g_v)
    pltpu.sync_copy(g_v, xg_hbm.at[pos_v.at[0]])


def _s1c_call(g_flat, pos):
    k = pl.kernel(
        _s1c_body,
        out_type=jax.ShapeDtypeStruct((NP, H), jnp.float32),
        mesh=_mesh(),
        scratch_types=[pltpu.VMEM((1, CHUNK), jnp.int32),
                       pltpu.VMEM((CHUNK, H), jnp.float32)],
        compiler_params=_sc_params(),
    )
    return k(g_flat, pos)


def _s2_body(yx_hbm, pos_hbm, mid_hbm, pos_v, y_v):
    wid = _wid()
    pltpu.sync_copy(pos_hbm.at[wid], pos_v)
    pltpu.sync_copy(yx_hbm.at[pos_v.at[0]], y_v)
    pltpu.sync_copy(y_v, mid_hbm.at[wid])


def _s2_call(yx, pos):
    k = pl.kernel(
        _s2_body,
        out_type=jax.ShapeDtypeStruct((NW, CHUNK, 256), jnp.float32),
        mesh=_mesh(),
        scratch_types=[pltpu.VMEM((1, CHUNK), jnp.int32),
                       pltpu.VMEM((CHUNK, 256), jnp.float32)],
        compiler_params=_sc_params(),
    )
    return k(yx, pos)


def _pre_body(x_ref, w1_ref, b1_ref, w2_ref, b2_ref, w3_ref, b3_ref,
              wih_ref, bih_ref, gx_ref):
    x = x_ref[...]
    a = _leaky(_mm(x, w1_ref[...]) + b1_ref[...])
    a = _leaky(_mm(a, w2_ref[...]) + b2_ref[...])
    inp = _mm(a, w3_ref[...]) + b3_ref[...]
    gx_ref[...] = _mm(inp, wih_ref[...]) + bih_ref[...]


def _pre_call(x, w1, b1, w2, b2, w3, b3, wih_t, bih):
    mblk = 256
    full = lambda s: pl.BlockSpec(s, lambda i: (0, 0))
    return pl.pallas_call(
        _pre_body,
        grid=(N // mblk,),
        in_specs=[
            pl.BlockSpec((mblk, KPAD), lambda i: (i, 0)),
            full((KPAD, H)), full((1, H)),
            full((H, H)), full((1, H)),
            full((H, H)), full((1, H)),
            full((H, 3 * H)), full((1, 3 * H)),
        ],
        out_specs=pl.BlockSpec((mblk, 3 * H), lambda i: (i, 0)),
        out_shape=jax.ShapeDtypeStruct((N, 3 * H), jnp.float32),
    )(x, w1, b1, w2, b2, w3, b3, wih_t, bih)


def _gru_body(gx_ref, code_ref, whh_ref, bhh_ref, wcc_ref,
              out_ref, cc_ref, h_ref):
    t = pl.program_id(0)

    @pl.when(t == 0)
    def _():
        h_ref[...] = code_ref[...]
        cc_ref[...] = _mm(code_ref[...], wcc_ref[...])

    h = h_ref[...]
    gh = _mm(h, whh_ref[...]) + bhh_ref[...]
    gx = gx_ref[:, 0, 0, :]
    r = jax.nn.sigmoid(gx[:, 0:H] + gh[:, 0:H])
    z = jax.nn.sigmoid(gx[:, H:2 * H] + gh[:, H:2 * H])
    n = jnp.tanh(gx[:, 2 * H:] + r * gh[:, 2 * H:])
    hn = (1.0 - z) * n + z * h
    h_ref[...] = hn
    out_ref[:, 0, 0, :] = hn


def _gru_call(gx, code, whh_t, bhh, wcc):
    return pl.pallas_call(
        _gru_body,
        grid=(T,),
        in_specs=[
            pl.BlockSpec((B, 1, 1, 3 * H), lambda t: (0, t, 0, 0)),
            pl.BlockSpec((B, H), lambda t: (0, 0)),
            pl.BlockSpec((H, 3 * H), lambda t: (0, 0)),
            pl.BlockSpec((1, 3 * H), lambda t: (0, 0)),
            pl.BlockSpec((H, 768), lambda t: (0, 0)),
        ],
        out_specs=[
            pl.BlockSpec((B, 1, 1, H), lambda t: (0, t, 0, 0)),
            pl.BlockSpec((B, 768), lambda t: (0, 0)),
        ],
        out_shape=[
            jax.ShapeDtypeStruct((B, T, 1, H), jnp.float32),
            jax.ShapeDtypeStruct((B, 768), jnp.float32),
        ],
        scratch_shapes=[pltpu.VMEM((B, H), jnp.float32)],
        compiler_params=pltpu.CompilerParams(
            dimension_semantics=("arbitrary",)),
    )(gx, code, whh_t, bhh, wcc)


def _grp_body(blk_ref, xg_ref, xgt_ref,
              w1g_ref, w1t_ref, b1_ref,
              w2d_ref, b2d_ref, w2f_ref, b2f_ref, w2b_ref, b2b_ref,
              w3d_ref, w3f_ref, w3b_ref, b3_ref, p_ref, out_ref):
    m = xg_ref.shape[0]
    term = _mm(xgt_ref[...], w1t_ref[0])
    z = jnp.zeros((m, 256), jnp.float32)
    h1 = _leaky(_mm(xg_ref[...], w1g_ref[0])
                + jnp.concatenate([z, term, z], axis=1) + b1_ref[0])
    h2d = _leaky(_mm(h1[:, 0:256], w2d_ref[0]) + b2d_ref[0])
    h2f = _leaky(_mm(h1[:, 256:512], w2f_ref[0]) + b2f_ref[0])
    h2b = _leaky(_mm(h1[:, 512:768], w2b_ref[0]) + b2b_ref[0])
    y = (_mm(h2d, w3d_ref[0]) + _mm(h2f, w3f_ref[0]) + _mm(h2b, w3b_ref[0])
         + b3_ref[0])
    out_ref[...] = _mm(y, p_ref[0])


def _grp_call(blk, xg, xgt, ew):
    ex = lambda s: pl.BlockSpec((1,) + s, lambda k, b: (b[k], 0, 0))
    grid_spec = pltpu.PrefetchScalarGridSpec(
        num_scalar_prefetch=1,
        grid=(NBLK,),
        in_specs=[
            pl.BlockSpec((128, H), lambda k, b: (k, 0)),
            pl.BlockSpec((128, KPAD), lambda k, b: (k, 0)),
            ex((H, 768)), ex((KPAD, 256)), ex((1, 768)),
            ex((256, 128)), ex((1, 128)),
            ex((256, 128)), ex((1, 128)),
            ex((256, 128)), ex((1, 128)),
            ex((128, 32)), ex((128, 32)), ex((128, 32)), ex((1, 32)),
            ex((32, 256)),
        ],
        out_specs=pl.BlockSpec((128, 256), lambda k, b: (k, 0)),
    )
    return pl.pallas_call(
        _grp_body,
        grid_spec=grid_spec,
        out_shape=jax.ShapeDtypeStruct((NP, 256), jnp.float32),
    )(blk, xg, xgt, *ew)


def _tail_body(g_ref, sel_ref, cc_ref, fu_ref, ch_ref, nc_ref,
               func_ref, child_ref, ncod_ref):
    g = g_ref[...]
    ccboth = _mm(sel_ref[...], cc_ref[...])

    fw1, fb1, fw2, fb2, fw3, fb3 = (fu_ref[i][...] for i in range(6))
    f = _leaky(_mm(g, fw1) + fb1)
    f = _leaky(_mm(f, fw2) + fb2)
    func_ref[...] = _mm(f, fw3) + fb3

    cwg, cb1, cw2, cb2, cw3, cb3 = (ch_ref[i][...] for i in range(6))
    c = _leaky(_mm(g, cwg) + ccboth[:, :256] + cb1)
    c = _leaky(_mm(c, cw2) + cb2)
    child_ref[...] = _mm(c, cw3) + cb3

    nwg, nb1, nw2, nb2, nw3, nb3 = (nc_ref[i][...] for i in range(6))
    nn = _leaky(_mm(g, nwg) + ccboth[:, 256:] + nb1)
    nn = _leaky(_mm(nn, nw2) + nb2)
    ncod_ref[...] = _mm(nn, nw3) + nb3


def _tail_call(g, sel, cc, fu, ch, nc):
    mblk = 256
    full = lambda a: pl.BlockSpec(a.shape, lambda i: tuple(0 for _ in a.shape))
    return pl.pallas_call(
        _tail_body,
        grid=(N // mblk,),
        in_specs=[
            pl.BlockSpec((mblk, H), lambda i: (i, 0)),
            pl.BlockSpec((mblk, B), lambda i: (i, 0)),
            pl.BlockSpec((B, 768), lambda i: (0, 0)),
            [full(a) for a in fu],
            [full(a) for a in ch],
            [full(a) for a in nc],
        ],
        out_specs=[
            pl.BlockSpec((mblk, NF), lambda i: (i, 0)),
            pl.BlockSpec((mblk, 4), lambda i: (i, 0)),
            pl.BlockSpec((mblk, 4 * H), lambda i: (i, 0)),
        ],
        out_shape=[
            jax.ShapeDtypeStruct((N, NF), jnp.float32),
            jax.ShapeDtypeStruct((N, 4), jnp.float32),
            jax.ShapeDtypeStruct((N, 4 * H), jnp.float32),
        ],
    )(g, sel, cc, fu, ch, nc)


def _row(b):
    return b.reshape(1, -1)


def _expert_weights(p):
    disc, fnet, bnet = p["disc"], p["fnet"], p["bnet"]
    hw = H // 2
    z = lambda *s: jnp.zeros(s, jnp.float32)
    w1g, w1t, b1 = [], [], []
    w2d, w2f, w2b = [], [], []
    b2d, b2f, b2b = [], [], []
    w3d, w3f, w3b, b3 = [], [], [], []
    for i in range(NF):
        fW1 = fnet[i][0]["W"]
        w1g.append(jnp.concatenate(
            [disc[i][0]["W"], fW1[:H], bnet[i][0]["W"]], axis=1))
        w1t.append(jnp.concatenate([
            z(NF + 32 * i, hw), fW1[H:H + 24], z(TL - 32 * i - 32, hw),
            fW1[H + 24:], z(KPAD - TL - 3, hw)], axis=0))
        b1.append(jnp.concatenate(
            [disc[i][0]["b"], fnet[i][0]["b"], bnet[i][0]["b"]]).reshape(1, -1))
        w2d.append(disc[i][1]["W"])
        w2f.append(fnet[i][1]["W"])
        w2b.append(bnet[i][1]["W"])
        b2d.append(_row(disc[i][1]["b"]))
        b2f.append(_row(fnet[i][1]["b"]))
        b2b.append(_row(bnet[i][1]["b"]))
        w3d.append(jnp.concatenate([disc[i][2]["W"], z(128, 8)], axis=1))
        w3f.append(jnp.concatenate(
            [z(128, 24), fnet[i][2]["W"], z(128, 2)], axis=1))
        w3b.append(jnp.concatenate([z(128, 30), bnet[i][2]["W"]], axis=1))
        b3.append(jnp.concatenate(
            [disc[i][2]["b"], fnet[i][2]["b"], bnet[i][2]["b"]]).reshape(1, -1))
    pm = np.zeros((NF, 32, 256), np.float32)
    for e in range(NF):
        pm[e, :, 32 * e:32 * e + 32] = np.eye(32, dtype=np.float32)
    st = lambda xs: jnp.stack(xs)
    return [st(w1g), st(w1t), st(b1),
            st(w2d), st(b2d), st(w2f), st(b2f), st(w2b), st(b2b),
            st(w3d), st(w3f), st(w3b), st(b3), jnp.asarray(pm)]


def kernel(inp_seq, code, _bb_dims, _hier_ind, gt_seq, params):
    b, t, tl = inp_seq.shape

    hier_oh = jax.nn.one_hot(_hier_ind, MAX_DEPTH, dtype=jnp.float32)
    feat = jnp.concatenate(
        [_bb_dims, hier_oh, jnp.zeros((B, KPAD - TL - 7), jnp.float32)],
        axis=1)
    x_flat = jnp.concatenate(
        [inp_seq.reshape(N, TL),
         jnp.broadcast_to(feat[:, None], (B, T, KPAD - TL)).reshape(N, -1)],
        axis=1)

    gt_flat = gt_seq.reshape(N, TL)
    gt8 = gt_flat[:, :NF]
    bb_flat = jnp.broadcast_to(_bb_dims[:, None], (B, T, 3)).reshape(N, 3)
    gtbb = jnp.concatenate(
        [gt_flat, bb_flat, jnp.zeros((N, KPAD - TL - 3), jnp.float32)],
        axis=1)
    sel = (jnp.arange(N, dtype=jnp.int32)[:, None] // T
           == jnp.arange(B, dtype=jnp.int32)[None, :]).astype(jnp.float32)

    p = params
    inp_net = p["inp_net"]
    w1 = jnp.concatenate(
        [inp_net[0]["W"], jnp.zeros((KPAD - TL - 7, H), jnp.float32)], axis=0)

    cmd, hist = _s1a_call(gt8)
    pos, xgtbb, blk = _s1b_call(cmd, hist, gtbb)

    gru = p["gru"]
    gx = _pre_call(x_flat, w1, _row(inp_net[0]["b"]),
                   inp_net[1]["W"], _row(inp_net[1]["b"]),
                   inp_net[2]["W"], _row(inp_net[2]["b"]),
                   gru["W_ih"].T, _row(gru["b_ih"]))
    wcc = jnp.concatenate(
        [p["child_net"][0]["W"][H:], p["next_code_net"][0]["W"][H:]], axis=1)
    gru_out, cc = _gru_call(gx.reshape(B, T, 1, 3 * H), code, gru["W_hh"].T,
                            _row(gru["b_hh"]), wcc)
    g_flat = gru_out.reshape(N, H)

    xg = _s1c_call(g_flat, pos)
    ew = _expert_weights(p)
    yx = _grp_call(blk, xg, xgtbb, ew)
    mid = _s2_call(yx, pos)

    fu = [p["func_net"][0]["W"], _row(p["func_net"][0]["b"]),
          p["func_net"][1]["W"], _row(p["func_net"][1]["b"]),
          p["func_net"][2]["W"], _row(p["func_net"][2]["b"])]
    ch = [p["child_net"][0]["W"][:H],
          _row(p["child_net"][0]["b"]),
          p["child_net"][1]["W"], _row(p["child_net"][1]["b"]),
          p["child_net"][2]["W"], _row(p["child_net"][2]["b"])]
    nc = [p["next_code_net"][0]["W"][:H],
          _row(p["next_code_net"][0]["b"]),
          p["next_code_net"][1]["W"], _row(p["next_code_net"][1]["b"]),
          p["next_code_net"][2]["W"], _row(p["next_code_net"][2]["b"])]
    func8, child_f, ncod_f = _tail_call(g_flat, sel, cc, fu, ch, nc)

    out_f = jnp.concatenate([func8, mid.reshape(N, 256)], axis=1)
    out = out_f.reshape(b, t, tl)
    child = child_f.reshape(b, t, 4)
    ncod = ncod_f.reshape(b, t, 4, H)
    return (out, ncod, child)


# speedup vs baseline: 1.1384x; 1.1384x over previous
import dataclasses
import functools

import jax
import jax.numpy as jnp
import numpy as np
from jax import lax
from jax.experimental import pallas as pl
from jax.experimental.pallas import tpu as pltpu
from jax.experimental.pallas import tpu_sc as plsc

MAX_DEPTH = 4
NF = 8
B, T, H, TL = 32, 64, 512, 264
N = B * T
KPAD = 384
NP = 3072
NBLK = NP // 128
NW = 32
CHUNK = N // NW

@functools.cache
def _mesh():
    return plsc.VectorSubcoreMesh(core_axis_name="c", subcore_axis_name="s")


def _sc_params():
    cp = pltpu.CompilerParams()
    if "needs_layout_passes" in pltpu.CompilerParams.__dataclass_fields__:
        cp = dataclasses.replace(cp, needs_layout_passes=False)
    return cp


def _leaky(x):
    return jnp.where(x >= 0, x, 0.2 * x)


def _mm(a, b):
    return a @ b


def _wid():
    return lax.axis_index("s") * 2 + lax.axis_index("c")


def _s1a_body(gt8_hbm, cmd_hbm, hist_hbm, gt8_v, cmd_v, hist_v):
    wid = _wid()
    base = wid * CHUNK
    pltpu.sync_copy(gt8_hbm.at[pl.ds(base, CHUNK)], gt8_v)
    lane = lax.iota(jnp.int32, 16)
    hist = jnp.zeros((16,), jnp.int32)
    for g in range(CHUNK // 16):
        rows = lane + g * 16
        best = plsc.load_gather(gt8_v, [rows, jnp.zeros((16,), jnp.int32)])
        bi = jnp.zeros((16,), jnp.int32)
        for j in range(1, NF):
            colj = plsc.load_gather(
                gt8_v, [rows, jnp.full((16,), j, jnp.int32)])
            m = colj > best
            bi = jnp.where(m, j, bi)
            best = jnp.where(m, colj, best)
        cmd_v[pl.ds(g * 16, 16)] = bi
        for e in range(NF):
            cnt = plsc.all_reduce_population_count(bi == e)
            hist = hist + jnp.where(lane == e, cnt, 0)
    hist_v[...] = hist
    pltpu.sync_copy(cmd_v, cmd_hbm.at[pl.ds(base, CHUNK)])
    pltpu.sync_copy(hist_v, hist_hbm.at[wid])


def _s1a_call(gt8):
    k = pl.kernel(
        _s1a_body,
        out_type=[jax.ShapeDtypeStruct((N,), jnp.int32),
                  jax.ShapeDtypeStruct((NW, 16), jnp.int32)],
        mesh=_mesh(),
        scratch_types=[pltpu.VMEM((CHUNK, NF), jnp.float32),
                       pltpu.VMEM((CHUNK,), jnp.int32),
                       pltpu.VMEM((16,), jnp.int32)],
        compiler_params=_sc_params(),
    )
    return k(gt8)


def _s1b_body(cmd_hbm, hist_hbm, gtbb_hbm,
              pos_hbm, xgtbb_hbm, blk_hbm,
              cmd_v, hist_v, pos_v, gtbb_v, blk_v):
    wid = _wid()
    base = wid * CHUNK
    lane = lax.iota(jnp.int32, 16)
    pltpu.sync_copy(cmd_hbm.at[pl.ds(base, CHUNK)], cmd_v)
    pltpu.sync_copy(hist_hbm, hist_v)
    pltpu.sync_copy(gtbb_hbm.at[pl.ds(base, CHUNK)], gtbb_v)

    tot = jnp.zeros((16,), jnp.int32)
    prefix = jnp.zeros((16,), jnp.int32)
    for w in range(NW):
        row = hist_v[w]
        tot = tot + row
        prefix = prefix + row * ((w < wid).astype(jnp.int32))
    pc = ((tot + 127) >> 7) << 7
    po = jnp.cumsum(pc) - pc
    basev = po + prefix

    for g in range(CHUNK // 16):
        cm = cmd_v[pl.ds(g * 16, 16)]
        dest = jnp.zeros((16,), jnp.int32)
        for e in range(NF):
            m = cm == e
            rk = jnp.cumsum(m.astype(jnp.int32)) - 1
            be = jnp.sum(jnp.where(lane == e, basev, 0))
            dest = jnp.where(m, be + rk, dest)
            cnt = plsc.all_reduce_population_count(m)
            basev = basev + jnp.where(lane == e, cnt, 0)
        pos_v[0, pl.ds(g * 16, 16)] = dest

    pltpu.sync_copy(gtbb_v, xgtbb_hbm.at[pos_v.at[0]])
    pltpu.sync_copy(pos_v, pos_hbm.at[wid])

    @pl.when(wid == 0)
    def _():
        blo = jnp.zeros((16,), jnp.int32)
        bhi = jnp.zeros((16,), jnp.int32)
        k1 = lane * 128
        k2 = (lane + 16) * 128
        for e in range(NF):
            po_e = jnp.sum(jnp.where(lane == e, po, 0))
            pc_e = jnp.sum(jnp.where(lane == e, pc, 0))
            blo = jnp.where((k1 >= po_e) & (k1 < po_e + pc_e), e, blo)
            bhi = jnp.where((k2 >= po_e) & (k2 < po_e + pc_e), e, bhi)
        blk_v[pl.ds(0, 16)] = blo
        blk_v[pl.ds(16, 16)] = bhi
        pltpu.sync_copy(blk_v, blk_hbm)


def _s1b_call(cmd, hist, gtbb):
    k = pl.kernel(
        _s1b_body,
        out_type=[jax.ShapeDtypeStruct((NW, 1, CHUNK), jnp.int32),
                  jax.ShapeDtypeStruct((NP, KPAD), jnp.float32),
                  jax.ShapeDtypeStruct((NW,), jnp.int32)],
        mesh=_mesh(),
        scratch_types=[pltpu.VMEM((CHUNK,), jnp.int32),
                       pltpu.VMEM((NW, 16), jnp.int32),
                       pltpu.VMEM((1, CHUNK), jnp.int32),
                       pltpu.VMEM((CHUNK, KPAD), jnp.float32),
                       pltpu.VMEM((NW,), jnp.int32)],
        compiler_params=_sc_params(),
    )
    return k(cmd, hist, gtbb)


def _s1c_body(g_hbm, pos_hbm, xg_hbm, pos_v, g_v):
    wid = _wid()
    pltpu.sync_copy(pos_hbm.at[wid], pos_v)
    pltpu.sync_copy(g_hbm.at[pl.ds(wid * CHUNK, CHUNK)], g_v)
    pltpu.sync_copy(g_v, xg_hbm.at[pos_v.at[0]])


def _s1c_call(g_flat, pos):
    k = pl.kernel(
        _s1c_body,
        out_type=jax.ShapeDtypeStruct((NP, H), jnp.float32),
        mesh=_mesh(),
        scratch_types=[pltpu.VMEM((1, CHUNK), jnp.int32),
                       pltpu.VMEM((CHUNK, H), jnp.float32)],
        compiler_params=_sc_params(),
    )
    return k(g_flat, pos)


def _s2_body(yx_hbm, pos_hbm, mid_hbm, pos_v, y_v):
    wid = _wid()
    pltpu.sync_copy(pos_hbm.at[wid], pos_v)
    pltpu.sync_copy(yx_hbm.at[pos_v.at[0]], y_v)
    pltpu.sync_copy(y_v, mid_hbm.at[wid])


def _s2_call(yx, pos):
    k = pl.kernel(
        _s2_body,
        out_type=jax.ShapeDtypeStruct((NW, CHUNK, 256), jnp.float32),
        mesh=_mesh(),
        scratch_types=[pltpu.VMEM((1, CHUNK), jnp.int32),
                       pltpu.VMEM((CHUNK, 256), jnp.float32)],
        compiler_params=_sc_params(),
    )
    return k(yx, pos)


def _pre_body(x_ref, w1_ref, b1_ref, w2_ref, b2_ref, w3_ref, b3_ref,
              wih_ref, bih_ref, gx_ref):
    x = x_ref[...]
    a = _leaky(_mm(x, w1_ref[...]) + b1_ref[...])
    a = _leaky(_mm(a, w2_ref[...]) + b2_ref[...])
    inp = _mm(a, w3_ref[...]) + b3_ref[...]
    gx_ref[...] = _mm(inp, wih_ref[...]) + bih_ref[...]


def _pre_call(x, w1, b1, w2, b2, w3, b3, wih_t, bih):
    mblk = 256
    full = lambda s: pl.BlockSpec(s, lambda i: (0, 0))
    return pl.pallas_call(
        _pre_body,
        grid=(N // mblk,),
        in_specs=[
            pl.BlockSpec((mblk, KPAD), lambda i: (i, 0)),
            full((KPAD, H)), full((1, H)),
            full((H, H)), full((1, H)),
            full((H, H)), full((1, H)),
            full((H, 3 * H)), full((1, 3 * H)),
        ],
        out_specs=pl.BlockSpec((mblk, 3 * H), lambda i: (i, 0)),
        out_shape=jax.ShapeDtypeStruct((N, 3 * H), jnp.float32),
    )(x, w1, b1, w2, b2, w3, b3, wih_t, bih)


def _gru_body(gx_ref, code_ref, whh_ref, bhh_ref, wcc_ref,
              out_ref, cc_ref, h_ref):
    t = pl.program_id(0)

    @pl.when(t == 0)
    def _():
        h_ref[...] = code_ref[...]
        cc_ref[...] = _mm(code_ref[...], wcc_ref[...])

    h = h_ref[...]
    gh = _mm(h, whh_ref[...]) + bhh_ref[...]
    gx = gx_ref[0]
    r = jax.nn.sigmoid(gx[:, 0:H] + gh[:, 0:H])
    z = jax.nn.sigmoid(gx[:, H:2 * H] + gh[:, H:2 * H])
    n = jnp.tanh(gx[:, 2 * H:] + r * gh[:, 2 * H:])
    hn = (1.0 - z) * n + z * h
    h_ref[...] = hn
    out_ref[0] = hn


def _gru_call(gx, code, whh_t, bhh, wcc):
    return pl.pallas_call(
        _gru_body,
        grid=(T,),
        in_specs=[
            pl.BlockSpec((1, B, 3 * H), lambda t: (t, 0, 0)),
            pl.BlockSpec((B, H), lambda t: (0, 0)),
            pl.BlockSpec((H, 3 * H), lambda t: (0, 0)),
            pl.BlockSpec((1, 3 * H), lambda t: (0, 0)),
            pl.BlockSpec((H, 768), lambda t: (0, 0)),
        ],
        out_specs=[
            pl.BlockSpec((1, B, H), lambda t: (t, 0, 0)),
            pl.BlockSpec((B, 768), lambda t: (0, 0)),
        ],
        out_shape=[
            jax.ShapeDtypeStruct((T, B, H), jnp.float32),
            jax.ShapeDtypeStruct((B, 768), jnp.float32),
        ],
        scratch_shapes=[pltpu.VMEM((B, H), jnp.float32)],
        compiler_params=pltpu.CompilerParams(
            dimension_semantics=("arbitrary",)),
    )(gx, code, whh_t, bhh, wcc)


def _grp_body(blk_ref, xg_ref, xgt_ref,
              w1g_ref, w1t_ref, b1_ref,
              w2d_ref, b2d_ref, w2f_ref, b2f_ref, w2b_ref, b2b_ref,
              w3d_ref, w3f_ref, w3b_ref, b3_ref, p_ref, out_ref):
    m = xg_ref.shape[0]
    term = _mm(xgt_ref[...], w1t_ref[0])
    z = jnp.zeros((m, 256), jnp.float32)
    h1 = _leaky(_mm(xg_ref[...], w1g_ref[0])
                + jnp.concatenate([z, term, z], axis=1) + b1_ref[0])
    h2d = _leaky(_mm(h1[:, 0:256], w2d_ref[0]) + b2d_ref[0])
    h2f = _leaky(_mm(h1[:, 256:512], w2f_ref[0]) + b2f_ref[0])
    h2b = _leaky(_mm(h1[:, 512:768], w2b_ref[0]) + b2b_ref[0])
    y = (_mm(h2d, w3d_ref[0]) + _mm(h2f, w3f_ref[0]) + _mm(h2b, w3b_ref[0])
         + b3_ref[0])
    out_ref[...] = _mm(y, p_ref[0])


def _grp_call(blk, xg, xgt, ew):
    ex = lambda s: pl.BlockSpec((1,) + s, lambda k, b: (b[k], 0, 0))
    grid_spec = pltpu.PrefetchScalarGridSpec(
        num_scalar_prefetch=1,
        grid=(NBLK,),
        in_specs=[
            pl.BlockSpec((128, H), lambda k, b: (k, 0)),
            pl.BlockSpec((128, KPAD), lambda k, b: (k, 0)),
            ex((H, 768)), ex((KPAD, 256)), ex((1, 768)),
            ex((256, 128)), ex((1, 128)),
            ex((256, 128)), ex((1, 128)),
            ex((256, 128)), ex((1, 128)),
            ex((128, 32)), ex((128, 32)), ex((128, 32)), ex((1, 32)),
            ex((32, 256)),
        ],
        out_specs=pl.BlockSpec((128, 256), lambda k, b: (k, 0)),
    )
    return pl.pallas_call(
        _grp_body,
        grid_spec=grid_spec,
        out_shape=jax.ShapeDtypeStruct((NP, 256), jnp.float32),
    )(blk, xg, xgt, *ew)


def _tail_body(g_ref, cc_ref, fu_ref, ch_ref, nc_ref,
               func_ref, child_ref, ncod_ref):
    g = g_ref[...]
    reps = g_ref.shape[0] // B
    ccboth = jnp.concatenate([cc_ref[...]] * reps, axis=0)

    fw1, fb1, fw2, fb2, fw3, fb3 = (fu_ref[i][...] for i in range(6))
    f = _leaky(_mm(g, fw1) + fb1)
    f = _leaky(_mm(f, fw2) + fb2)
    func_ref[...] = _mm(f, fw3) + fb3

    cwg, cb1, cw2, cb2, cw3, cb3 = (ch_ref[i][...] for i in range(6))
    c = _leaky(_mm(g, cwg) + ccboth[:, :256] + cb1)
    c = _leaky(_mm(c, cw2) + cb2)
    child_ref[...] = _mm(c, cw3) + cb3

    nwg, nb1, nw2, nb2, nw3, nb3 = (nc_ref[i][...] for i in range(6))
    nn = _leaky(_mm(g, nwg) + ccboth[:, 256:] + nb1)
    nn = _leaky(_mm(nn, nw2) + nb2)
    ncod_ref[...] = _mm(nn, nw3) + nb3


def _tail_call(g, cc, fu, ch, nc):
    mblk = 256
    full = lambda a: pl.BlockSpec(a.shape, lambda i: tuple(0 for _ in a.shape))
    return pl.pallas_call(
        _tail_body,
        grid=(N // mblk,),
        in_specs=[
            pl.BlockSpec((mblk, H), lambda i: (i, 0)),
            pl.BlockSpec((B, 768), lambda i: (0, 0)),
            [full(a) for a in fu],
            [full(a) for a in ch],
            [full(a) for a in nc],
        ],
        out_specs=[
            pl.BlockSpec((mblk, NF), lambda i: (i, 0)),
            pl.BlockSpec((mblk, 4), lambda i: (i, 0)),
            pl.BlockSpec((mblk, 4 * H), lambda i: (i, 0)),
        ],
        out_shape=[
            jax.ShapeDtypeStruct((N, NF), jnp.float32),
            jax.ShapeDtypeStruct((N, 4), jnp.float32),
            jax.ShapeDtypeStruct((N, 4 * H), jnp.float32),
        ],
    )(g, cc, fu, ch, nc)


def _row(b):
    return b.reshape(1, -1)


def _expert_weights(p):
    disc, fnet, bnet = p["disc"], p["fnet"], p["bnet"]
    hw = H // 2
    z = lambda *s: jnp.zeros(s, jnp.float32)
    w1g, w1t, b1 = [], [], []
    w2d, w2f, w2b = [], [], []
    b2d, b2f, b2b = [], [], []
    w3d, w3f, w3b, b3 = [], [], [], []
    for i in range(NF):
        fW1 = fnet[i][0]["W"]
        w1g.append(jnp.concatenate(
            [disc[i][0]["W"], fW1[:H], bnet[i][0]["W"]], axis=1))
        w1t.append(jnp.concatenate([
            z(NF + 32 * i, hw), fW1[H:H + 24], z(TL - 32 * i - 32, hw),
            fW1[H + 24:], z(KPAD - TL - 3, hw)], axis=0))
        b1.append(jnp.concatenate(
            [disc[i][0]["b"], fnet[i][0]["b"], bnet[i][0]["b"]]).reshape(1, -1))
        w2d.append(disc[i][1]["W"])
        w2f.append(fnet[i][1]["W"])
        w2b.append(bnet[i][1]["W"])
        b2d.append(_row(disc[i][1]["b"]))
        b2f.append(_row(fnet[i][1]["b"]))
        b2b.append(_row(bnet[i][1]["b"]))
        w3d.append(jnp.concatenate([disc[i][2]["W"], z(128, 8)], axis=1))
        w3f.append(jnp.concatenate(
            [z(128, 24), fnet[i][2]["W"], z(128, 2)], axis=1))
        w3b.append(jnp.concatenate([z(128, 30), bnet[i][2]["W"]], axis=1))
        b3.append(jnp.concatenate(
            [disc[i][2]["b"], fnet[i][2]["b"], bnet[i][2]["b"]]).reshape(1, -1))
    pm = np.zeros((NF, 32, 256), np.float32)
    for e in range(NF):
        pm[e, :, 32 * e:32 * e + 32] = np.eye(32, dtype=np.float32)
    st = lambda xs: jnp.stack(xs)
    return [st(w1g), st(w1t), st(b1),
            st(w2d), st(b2d), st(w2f), st(b2f), st(w2b), st(b2b),
            st(w3d), st(w3f), st(w3b), st(b3), jnp.asarray(pm)]


def kernel(inp_seq, code, _bb_dims, _hier_ind, gt_seq, params):
    b, t, tl = inp_seq.shape

    x_t = jnp.swapaxes(inp_seq, 0, 1)
    hier_oh = jax.nn.one_hot(_hier_ind, MAX_DEPTH, dtype=jnp.float32)
    feat = jnp.concatenate(
        [_bb_dims, hier_oh, jnp.zeros((B, KPAD - TL - 7), jnp.float32)],
        axis=1)
    x_cat = jnp.concatenate(
        [x_t, jnp.broadcast_to(feat[None], (T, B, KPAD - TL))], axis=2)
    x_flat = x_cat.reshape(N, KPAD)

    gt_flat = jnp.swapaxes(gt_seq, 0, 1).reshape(N, TL)
    gt8 = gt_flat[:, :NF]
    bb_flat = jnp.broadcast_to(_bb_dims[None], (T, B, 3)).reshape(N, 3)
    gtbb = jnp.concatenate(
        [gt_flat, bb_flat, jnp.zeros((N, KPAD - TL - 3), jnp.float32)],
        axis=1)

    p = params
    inp_net = p["inp_net"]
    w1 = jnp.concatenate(
        [inp_net[0]["W"], jnp.zeros((KPAD - TL - 7, H), jnp.float32)], axis=0)

    cmd, hist = _s1a_call(gt8)
    pos, xgtbb, blk = _s1b_call(cmd, hist, gtbb)

    gru = p["gru"]
    gx = _pre_call(x_flat, w1, _row(inp_net[0]["b"]),
                   inp_net[1]["W"], _row(inp_net[1]["b"]),
                   inp_net[2]["W"], _row(inp_net[2]["b"]),
                   gru["W_ih"].T, _row(gru["b_ih"]))
    wcc = jnp.concatenate(
        [p["child_net"][0]["W"][H:], p["next_code_net"][0]["W"][H:]], axis=1)
    gru_out, cc = _gru_call(gx.reshape(T, B, 3 * H), code, gru["W_hh"].T,
                            _row(gru["b_hh"]), wcc)
    g_flat = gru_out.reshape(N, H)

    xg = _s1c_call(g_flat, pos)
    ew = _expert_weights(p)
    yx = _grp_call(blk, xg, xgtbb, ew)
    mid = _s2_call(yx, pos)

    fu = [p["func_net"][0]["W"], _row(p["func_net"][0]["b"]),
          p["func_net"][1]["W"], _row(p["func_net"][1]["b"]),
          p["func_net"][2]["W"], _row(p["func_net"][2]["b"])]
    ch = [p["child_net"][0]["W"][:H],
          _row(p["child_net"][0]["b"]),
          p["child_net"][1]["W"], _row(p["child_net"][1]["b"]),
          p["child_net"][2]["W"], _row(p["child_net"][2]["b"])]
    nc = [p["next_code_net"][0]["W"][:H],
          _row(p["next_code_net"][0]["b"]),
          p["next_code_net"][1]["W"], _row(p["next_code_net"][1]["b"]),
          p["next_code_net"][2]["W"], _row(p["next_code_net"][2]["b"])]
    func8, child_f, ncod_f = _tail_call(g_flat, cc, fu, ch, nc)

    out_f = jnp.concatenate([func8, mid.reshape(N, 256)], axis=1)
    out = jnp.swapaxes(out_f.reshape(T, B, TL), 0, 1)
    child = jnp.swapaxes(child_f.reshape(T, B, 4), 0, 1)
    ncod = jnp.swapaxes(ncod_f.reshape(T, B, 4 * H), 0, 1).reshape(b, t, 4, H)
    return (out, ncod, child)


# speedup vs baseline: 1.1736x; 1.0310x over previous
import dataclasses
import functools

import jax
import jax.numpy as jnp
import numpy as np
from jax import lax
from jax.experimental import pallas as pl
from jax.experimental.pallas import tpu as pltpu
from jax.experimental.pallas import tpu_sc as plsc

MAX_DEPTH = 4
NF = 8
B, T, H, TL = 32, 64, 512, 264
N = B * T
KPAD = 384
NP = 4096
CAP = 256
NBLK = NP // CAP
NW = 32
CHUNK = N // NW

@functools.cache
def _mesh():
    return plsc.VectorSubcoreMesh(core_axis_name="c", subcore_axis_name="s")


def _sc_params():
    cp = pltpu.CompilerParams()
    if "needs_layout_passes" in pltpu.CompilerParams.__dataclass_fields__:
        cp = dataclasses.replace(cp, needs_layout_passes=False)
    return cp


def _leaky(x):
    return jnp.where(x >= 0, x, 0.2 * x)


def _mm(a, b):
    return a @ b


def _wid():
    return lax.axis_index("s") * 2 + lax.axis_index("c")


def _s1a_body(gt8_hbm, cmd_hbm, hist_hbm, gt8_v, cmd_v, hist_v):
    wid = _wid()
    base = wid * CHUNK
    pltpu.sync_copy(gt8_hbm.at[pl.ds(base, CHUNK)], gt8_v)
    lane = lax.iota(jnp.int32, 16)
    hist = jnp.zeros((16,), jnp.int32)
    for g in range(CHUNK // 16):
        rows = lane + g * 16
        best = plsc.load_gather(gt8_v, [rows, jnp.zeros((16,), jnp.int32)])
        bi = jnp.zeros((16,), jnp.int32)
        for j in range(1, NF):
            colj = plsc.load_gather(
                gt8_v, [rows, jnp.full((16,), j, jnp.int32)])
            m = colj > best
            bi = jnp.where(m, j, bi)
            best = jnp.where(m, colj, best)
        cmd_v[pl.ds(g * 16, 16)] = bi
        for e in range(NF):
            cnt = plsc.all_reduce_population_count(bi == e)
            hist = hist + jnp.where(lane == e, cnt, 0)
    hist_v[...] = hist
    pltpu.sync_copy(cmd_v, cmd_hbm.at[pl.ds(base, CHUNK)])
    pltpu.sync_copy(hist_v, hist_hbm.at[wid])


def _s1a_call(gt8):
    k = pl.kernel(
        _s1a_body,
        out_type=[jax.ShapeDtypeStruct((N,), jnp.int32),
                  jax.ShapeDtypeStruct((NW, 16), jnp.int32)],
        mesh=_mesh(),
        scratch_types=[pltpu.VMEM((CHUNK, NF), jnp.float32),
                       pltpu.VMEM((CHUNK,), jnp.int32),
                       pltpu.VMEM((16,), jnp.int32)],
        compiler_params=_sc_params(),
    )
    return k(gt8)


def _s1b_body(cmd_hbm, hist_hbm, gtbb_hbm,
              pos_hbm, xgtbb_hbm, blk_hbm,
              cmd_v, hist_v, pos_v, gtbb_v, blk_v):
    wid = _wid()
    base = wid * CHUNK
    lane = lax.iota(jnp.int32, 16)
    pltpu.sync_copy(cmd_hbm.at[pl.ds(base, CHUNK)], cmd_v)
    pltpu.sync_copy(hist_hbm, hist_v)
    pltpu.sync_copy(gtbb_hbm.at[pl.ds(base, CHUNK)], gtbb_v)

    tot = jnp.zeros((16,), jnp.int32)
    prefix = jnp.zeros((16,), jnp.int32)
    for w in range(NW):
        row = hist_v[w]
        tot = tot + row
        prefix = prefix + row * ((w < wid).astype(jnp.int32))
    pc = ((tot + CAP - 1) >> 8) << 8
    po = jnp.cumsum(pc) - pc
    basev = po + prefix

    for g in range(CHUNK // 16):
        cm = cmd_v[pl.ds(g * 16, 16)]
        dest = jnp.zeros((16,), jnp.int32)
        for e in range(NF):
            m = cm == e
            rk = jnp.cumsum(m.astype(jnp.int32)) - 1
            be = jnp.sum(jnp.where(lane == e, basev, 0))
            dest = jnp.where(m, be + rk, dest)
            cnt = plsc.all_reduce_population_count(m)
            basev = basev + jnp.where(lane == e, cnt, 0)
        pos_v[0, pl.ds(g * 16, 16)] = dest

    pltpu.sync_copy(gtbb_v, xgtbb_hbm.at[pos_v.at[0]])
    pltpu.sync_copy(pos_v, pos_hbm.at[wid])

    @pl.when(wid == 0)
    def _():
        blo = jnp.zeros((16,), jnp.int32)
        k1 = lane * CAP
        for e in range(NF):
            po_e = jnp.sum(jnp.where(lane == e, po, 0))
            pc_e = jnp.sum(jnp.where(lane == e, pc, 0))
            blo = jnp.where((k1 >= po_e) & (k1 < po_e + pc_e), e, blo)
        blk_v[pl.ds(0, 16)] = blo
        blk_v[pl.ds(16, 16)] = jnp.zeros((16,), jnp.int32)
        pltpu.sync_copy(blk_v, blk_hbm)


def _s1b_call(cmd, hist, gtbb):
    k = pl.kernel(
        _s1b_body,
        out_type=[jax.ShapeDtypeStruct((NW, 1, CHUNK), jnp.int32),
                  jax.ShapeDtypeStruct((NP, KPAD), jnp.float32),
                  jax.ShapeDtypeStruct((NW,), jnp.int32)],
        mesh=_mesh(),
        scratch_types=[pltpu.VMEM((CHUNK,), jnp.int32),
                       pltpu.VMEM((NW, 16), jnp.int32),
                       pltpu.VMEM((1, CHUNK), jnp.int32),
                       pltpu.VMEM((CHUNK, KPAD), jnp.float32),
                       pltpu.VMEM((NW,), jnp.int32)],
        compiler_params=_sc_params(),
    )
    return k(cmd, hist, gtbb)


def _s1c_body(g_hbm, pos_hbm, xg_hbm, pos_v, g_v):
    wid = _wid()
    pltpu.sync_copy(pos_hbm.at[wid], pos_v)
    pltpu.sync_copy(g_hbm.at[pl.ds(wid * CHUNK, CHUNK)], g_v)
    pltpu.sync_copy(g_v, xg_hbm.at[pos_v.at[0]])


def _s1c_call(g_flat, pos):
    k = pl.kernel(
        _s1c_body,
        out_type=jax.ShapeDtypeStruct((NP, H), jnp.float32),
        mesh=_mesh(),
        scratch_types=[pltpu.VMEM((1, CHUNK), jnp.int32),
                       pltpu.VMEM((CHUNK, H), jnp.float32)],
        compiler_params=_sc_params(),
    )
    return k(g_flat, pos)


def _s2_body(yx_hbm, pos_hbm, mid_hbm, pos_v, y_v):
    wid = _wid()
    pltpu.sync_copy(pos_hbm.at[wid], pos_v)
    pltpu.sync_copy(yx_hbm.at[pos_v.at[0]], y_v)
    pltpu.sync_copy(y_v, mid_hbm.at[wid])


def _s2_call(yx, pos):
    k = pl.kernel(
        _s2_body,
        out_type=jax.ShapeDtypeStruct((NW, CHUNK, 256), jnp.float32),
        mesh=_mesh(),
        scratch_types=[pltpu.VMEM((1, CHUNK), jnp.int32),
                       pltpu.VMEM((CHUNK, 256), jnp.float32)],
        compiler_params=_sc_params(),
    )
    return k(yx, pos)


def _pre_body(x_ref, w1_ref, b1_ref, w2_ref, b2_ref, w3_ref, b3_ref,
              wih_ref, bih_ref, gx_ref):
    x = x_ref[...]
    a = _leaky(_mm(x, w1_ref[...]) + b1_ref[...])
    a = _leaky(_mm(a, w2_ref[...]) + b2_ref[...])
    inp = _mm(a, w3_ref[...]) + b3_ref[...]
    gx_ref[...] = _mm(inp, wih_ref[...]) + bih_ref[...]


def _pre_call(x, w1, b1, w2, b2, w3, b3, wih_t, bih):
    mblk = 256
    full = lambda s: pl.BlockSpec(s, lambda i: (0, 0))
    return pl.pallas_call(
        _pre_body,
        grid=(N // mblk,),
        in_specs=[
            pl.BlockSpec((mblk, KPAD), lambda i: (i, 0)),
            full((KPAD, H)), full((1, H)),
            full((H, H)), full((1, H)),
            full((H, H)), full((1, H)),
            full((H, 3 * H)), full((1, 3 * H)),
        ],
        out_specs=pl.BlockSpec((mblk, 3 * H), lambda i: (i, 0)),
        out_shape=jax.ShapeDtypeStruct((N, 3 * H), jnp.float32),
    )(x, w1, b1, w2, b2, w3, b3, wih_t, bih)


def _gru_body(gx_ref, code_ref, whh_ref, bhh_ref, wcc_ref,
              out_ref, cc_ref, h_ref):
    t = pl.program_id(0)

    @pl.when(t == 0)
    def _():
        h_ref[...] = code_ref[...]
        cc_ref[...] = _mm(code_ref[...], wcc_ref[...])

    h = h_ref[...]
    gh = _mm(h, whh_ref[...]) + bhh_ref[...]
    gx = gx_ref[0]
    r = jax.nn.sigmoid(gx[:, 0:H] + gh[:, 0:H])
    z = jax.nn.sigmoid(gx[:, H:2 * H] + gh[:, H:2 * H])
    n = jnp.tanh(gx[:, 2 * H:] + r * gh[:, 2 * H:])
    hn = (1.0 - z) * n + z * h
    h_ref[...] = hn
    out_ref[0] = hn


def _gru_call(gx, code, whh_t, bhh, wcc):
    return pl.pallas_call(
        _gru_body,
        grid=(T,),
        in_specs=[
            pl.BlockSpec((1, B, 3 * H), lambda t: (t, 0, 0)),
            pl.BlockSpec((B, H), lambda t: (0, 0)),
            pl.BlockSpec((H, 3 * H), lambda t: (0, 0)),
            pl.BlockSpec((1, 3 * H), lambda t: (0, 0)),
            pl.BlockSpec((H, 768), lambda t: (0, 0)),
        ],
        out_specs=[
            pl.BlockSpec((1, B, H), lambda t: (t, 0, 0)),
            pl.BlockSpec((B, 768), lambda t: (0, 0)),
        ],
        out_shape=[
            jax.ShapeDtypeStruct((T, B, H), jnp.float32),
            jax.ShapeDtypeStruct((B, 768), jnp.float32),
        ],
        scratch_shapes=[pltpu.VMEM((B, H), jnp.float32)],
        compiler_params=pltpu.CompilerParams(
            dimension_semantics=("arbitrary",)),
    )(gx, code, whh_t, bhh, wcc)


def _grp_body(blk_ref, xg_ref, xgt_ref,
              w1g_ref, w1t_ref, b1_ref,
              w2d_ref, b2d_ref, w2f_ref, b2f_ref, w2b_ref, b2b_ref,
              w3d_ref, w3f_ref, w3b_ref, b3_ref, p_ref, out_ref):
    m = xg_ref.shape[0]
    term = _mm(xgt_ref[...], w1t_ref[0])
    z = jnp.zeros((m, 256), jnp.float32)
    h1 = _leaky(_mm(xg_ref[...], w1g_ref[0])
                + jnp.concatenate([z, term, z], axis=1) + b1_ref[0])
    h2d = _leaky(_mm(h1[:, 0:256], w2d_ref[0]) + b2d_ref[0])
    h2f = _leaky(_mm(h1[:, 256:512], w2f_ref[0]) + b2f_ref[0])
    h2b = _leaky(_mm(h1[:, 512:768], w2b_ref[0]) + b2b_ref[0])
    y = (_mm(h2d, w3d_ref[0]) + _mm(h2f, w3f_ref[0]) + _mm(h2b, w3b_ref[0])
         + b3_ref[0])
    out_ref[...] = _mm(y, p_ref[0])


def _grp_call(blk, xg, xgt, ew):
    ex = lambda s: pl.BlockSpec((1,) + s, lambda k, b: (b[k], 0, 0))
    grid_spec = pltpu.PrefetchScalarGridSpec(
        num_scalar_prefetch=1,
        grid=(NBLK,),
        in_specs=[
            pl.BlockSpec((CAP, H), lambda k, b: (k, 0)),
            pl.BlockSpec((CAP, KPAD), lambda k, b: (k, 0)),
            ex((H, 768)), ex((KPAD, 256)), ex((1, 768)),
            ex((256, 128)), ex((1, 128)),
            ex((256, 128)), ex((1, 128)),
            ex((256, 128)), ex((1, 128)),
            ex((128, 32)), ex((128, 32)), ex((128, 32)), ex((1, 32)),
            ex((32, 256)),
        ],
        out_specs=pl.BlockSpec((CAP, 256), lambda k, b: (k, 0)),
    )
    return pl.pallas_call(
        _grp_body,
        grid_spec=grid_spec,
        out_shape=jax.ShapeDtypeStruct((NP, 256), jnp.float32),
    )(blk, xg, xgt, *ew)


def _tail_body(g_ref, cc_ref, fu_ref, ch_ref, nc_ref,
               func_ref, child_ref, ncod_ref):
    g = g_ref[...]
    reps = g_ref.shape[0] // B
    ccboth = jnp.concatenate([cc_ref[...]] * reps, axis=0)

    fw1, fb1, fw2, fb2, fw3, fb3 = (fu_ref[i][...] for i in range(6))
    f = _leaky(_mm(g, fw1) + fb1)
    f = _leaky(_mm(f, fw2) + fb2)
    func_ref[...] = _mm(f, fw3) + fb3

    cwg, cb1, cw2, cb2, cw3, cb3 = (ch_ref[i][...] for i in range(6))
    c = _leaky(_mm(g, cwg) + ccboth[:, :256] + cb1)
    c = _leaky(_mm(c, cw2) + cb2)
    child_ref[...] = _mm(c, cw3) + cb3

    nwg, nb1, nw2, nb2, nw3, nb3 = (nc_ref[i][...] for i in range(6))
    nn = _leaky(_mm(g, nwg) + ccboth[:, 256:] + nb1)
    nn = _leaky(_mm(nn, nw2) + nb2)
    ncod_ref[...] = _mm(nn, nw3) + nb3


def _tail_call(g, cc, fu, ch, nc):
    mblk = 256
    full = lambda a: pl.BlockSpec(a.shape, lambda i: tuple(0 for _ in a.shape))
    return pl.pallas_call(
        _tail_body,
        grid=(N // mblk,),
        in_specs=[
            pl.BlockSpec((mblk, H), lambda i: (i, 0)),
            pl.BlockSpec((B, 768), lambda i: (0, 0)),
            [full(a) for a in fu],
            [full(a) for a in ch],
            [full(a) for a in nc],
        ],
        out_specs=[
            pl.BlockSpec((mblk, NF), lambda i: (i, 0)),
            pl.BlockSpec((mblk, 4), lambda i: (i, 0)),
            pl.BlockSpec((mblk, 4 * H), lambda i: (i, 0)),
        ],
        out_shape=[
            jax.ShapeDtypeStruct((N, NF), jnp.float32),
            jax.ShapeDtypeStruct((N, 4), jnp.float32),
            jax.ShapeDtypeStruct((N, 4 * H), jnp.float32),
        ],
    )(g, cc, fu, ch, nc)


def _row(b):
    return b.reshape(1, -1)


def _expert_weights(p):
    disc, fnet, bnet = p["disc"], p["fnet"], p["bnet"]
    hw = H // 2
    z = lambda *s: jnp.zeros(s, jnp.float32)
    w1g, w1t, b1 = [], [], []
    w2d, w2f, w2b = [], [], []
    b2d, b2f, b2b = [], [], []
    w3d, w3f, w3b, b3 = [], [], [], []
    for i in range(NF):
        fW1 = fnet[i][0]["W"]
        w1g.append(jnp.concatenate(
            [disc[i][0]["W"], fW1[:H], bnet[i][0]["W"]], axis=1))
        w1t.append(jnp.concatenate([
            z(NF + 32 * i, hw), fW1[H:H + 24], z(TL - 32 * i - 32, hw),
            fW1[H + 24:], z(KPAD - TL - 3, hw)], axis=0))
        b1.append(jnp.concatenate(
            [disc[i][0]["b"], fnet[i][0]["b"], bnet[i][0]["b"]]).reshape(1, -1))
        w2d.append(disc[i][1]["W"])
        w2f.append(fnet[i][1]["W"])
        w2b.append(bnet[i][1]["W"])
        b2d.append(_row(disc[i][1]["b"]))
        b2f.append(_row(fnet[i][1]["b"]))
        b2b.append(_row(bnet[i][1]["b"]))
        w3d.append(jnp.concatenate([disc[i][2]["W"], z(128, 8)], axis=1))
        w3f.append(jnp.concatenate(
            [z(128, 24), fnet[i][2]["W"], z(128, 2)], axis=1))
        w3b.append(jnp.concatenate([z(128, 30), bnet[i][2]["W"]], axis=1))
        b3.append(jnp.concatenate(
            [disc[i][2]["b"], fnet[i][2]["b"], bnet[i][2]["b"]]).reshape(1, -1))
    pm = np.zeros((NF, 32, 256), np.float32)
    for e in range(NF):
        pm[e, :, 32 * e:32 * e + 32] = np.eye(32, dtype=np.float32)
    st = lambda xs: jnp.stack(xs)
    return [st(w1g), st(w1t), st(b1),
            st(w2d), st(b2d), st(w2f), st(b2f), st(w2b), st(b2b),
            st(w3d), st(w3f), st(w3b), st(b3), jnp.asarray(pm)]


def kernel(inp_seq, code, _bb_dims, _hier_ind, gt_seq, params):
    b, t, tl = inp_seq.shape

    x_t = jnp.swapaxes(inp_seq, 0, 1)
    hier_oh = jax.nn.one_hot(_hier_ind, MAX_DEPTH, dtype=jnp.float32)
    feat = jnp.concatenate(
        [_bb_dims, hier_oh, jnp.zeros((B, KPAD - TL - 7), jnp.float32)],
        axis=1)
    x_cat = jnp.concatenate(
        [x_t, jnp.broadcast_to(feat[None], (T, B, KPAD - TL))], axis=2)
    x_flat = x_cat.reshape(N, KPAD)

    gt_flat = jnp.swapaxes(gt_seq, 0, 1).reshape(N, TL)
    gt8 = gt_flat[:, :NF]
    bb_flat = jnp.broadcast_to(_bb_dims[None], (T, B, 3)).reshape(N, 3)
    gtbb = jnp.concatenate(
        [gt_flat, bb_flat, jnp.zeros((N, KPAD - TL - 3), jnp.float32)],
        axis=1)

    p = params
    inp_net = p["inp_net"]
    w1 = jnp.concatenate(
        [inp_net[0]["W"], jnp.zeros((KPAD - TL - 7, H), jnp.float32)], axis=0)

    cmd, hist = _s1a_call(gt8)
    pos, xgtbb, blk = _s1b_call(cmd, hist, gtbb)

    gru = p["gru"]
    gx = _pre_call(x_flat, w1, _row(inp_net[0]["b"]),
                   inp_net[1]["W"], _row(inp_net[1]["b"]),
                   inp_net[2]["W"], _row(inp_net[2]["b"]),
                   gru["W_ih"].T, _row(gru["b_ih"]))
    wcc = jnp.concatenate(
        [p["child_net"][0]["W"][H:], p["next_code_net"][0]["W"][H:]], axis=1)
    gru_out, cc = _gru_call(gx.reshape(T, B, 3 * H), code, gru["W_hh"].T,
                            _row(gru["b_hh"]), wcc)
    g_flat = gru_out.reshape(N, H)

    xg = _s1c_call(g_flat, pos)
    ew = _expert_weights(p)
    yx = _grp_call(blk, xg, xgtbb, ew)
    mid = _s2_call(yx, pos)

    fu = [p["func_net"][0]["W"], _row(p["func_net"][0]["b"]),
          p["func_net"][1]["W"], _row(p["func_net"][1]["b"]),
          p["func_net"][2]["W"], _row(p["func_net"][2]["b"])]
    ch = [p["child_net"][0]["W"][:H],
          _row(p["child_net"][0]["b"]),
          p["child_net"][1]["W"], _row(p["child_net"][1]["b"]),
          p["child_net"][2]["W"], _row(p["child_net"][2]["b"])]
    nc = [p["next_code_net"][0]["W"][:H],
          _row(p["next_code_net"][0]["b"]),
          p["next_code_net"][1]["W"], _row(p["next_code_net"][1]["b"]),
          p["next_code_net"][2]["W"], _row(p["next_code_net"][2]["b"])]
    func8, child_f, ncod_f = _tail_call(g_flat, cc, fu, ch, nc)

    out_f = jnp.concatenate([func8, mid.reshape(N, 256)], axis=1)
    out = jnp.swapaxes(out_f.reshape(T, B, TL), 0, 1)
    child = jnp.swapaxes(child_f.reshape(T, B, 4), 0, 1)
    ncod = jnp.swapaxes(ncod_f.reshape(T, B, 4 * H), 0, 1).reshape(b, t, 4, H)
    return (out, ncod, child)


# speedup vs baseline: 1.2041x; 1.0260x over previous
import dataclasses
import functools

import jax
import jax.numpy as jnp
import numpy as np
from jax import lax
from jax.experimental import pallas as pl
from jax.experimental.pallas import tpu as pltpu
from jax.experimental.pallas import tpu_sc as plsc

MAX_DEPTH = 4
NF = 8
B, T, H, TL = 32, 64, 512, 264
N = B * T
KPAD = 384
NP = 4096
CAP = 256
NBLK = NP // CAP
NW = 32
CHUNK = N // NW

@functools.cache
def _mesh():
    return plsc.VectorSubcoreMesh(core_axis_name="c", subcore_axis_name="s")


def _sc_params():
    cp = pltpu.CompilerParams()
    if "needs_layout_passes" in pltpu.CompilerParams.__dataclass_fields__:
        cp = dataclasses.replace(cp, needs_layout_passes=False)
    return cp


def _leaky(x):
    return jnp.where(x >= 0, x, 0.2 * x)


def _mm(a, b):
    return a @ b


def _wid():
    return lax.axis_index("s") * 2 + lax.axis_index("c")


def _s1a_body(gt8_hbm, cmd_hbm, hist_hbm, gt8_v, cmd_v, hist_v):
    wid = _wid()
    base = wid * CHUNK
    pltpu.sync_copy(gt8_hbm.at[pl.ds(base, CHUNK)], gt8_v)
    lane = lax.iota(jnp.int32, 16)
    hist = jnp.zeros((16,), jnp.int32)
    for g in range(CHUNK // 16):
        rows = lane + g * 16
        best = plsc.load_gather(gt8_v, [rows, jnp.zeros((16,), jnp.int32)])
        bi = jnp.zeros((16,), jnp.int32)
        for j in range(1, NF):
            colj = plsc.load_gather(
                gt8_v, [rows, jnp.full((16,), j, jnp.int32)])
            m = colj > best
            bi = jnp.where(m, j, bi)
            best = jnp.where(m, colj, best)
        cmd_v[pl.ds(g * 16, 16)] = bi
        for e in range(NF):
            cnt = plsc.all_reduce_population_count(bi == e)
            hist = hist + jnp.where(lane == e, cnt, 0)
    hist_v[...] = hist
    pltpu.sync_copy(cmd_v, cmd_hbm.at[pl.ds(base, CHUNK)])
    pltpu.sync_copy(hist_v, hist_hbm.at[wid])


def _s1a_call(gt8):
    k = pl.kernel(
        _s1a_body,
        out_type=[jax.ShapeDtypeStruct((N,), jnp.int32),
                  jax.ShapeDtypeStruct((NW, 16), jnp.int32)],
        mesh=_mesh(),
        scratch_types=[pltpu.VMEM((CHUNK, NF), jnp.float32),
                       pltpu.VMEM((CHUNK,), jnp.int32),
                       pltpu.VMEM((16,), jnp.int32)],
        compiler_params=_sc_params(),
    )
    return k(gt8)


def _s1b_body(cmd_hbm, hist_hbm, gtbb_hbm,
              pos_hbm, xgtbb_hbm, blk_hbm,
              cmd_v, hist_v, pos_v, gtbb_v, blk_v):
    wid = _wid()
    base = wid * CHUNK
    lane = lax.iota(jnp.int32, 16)
    pltpu.sync_copy(cmd_hbm.at[pl.ds(base, CHUNK)], cmd_v)
    pltpu.sync_copy(hist_hbm, hist_v)
    pltpu.sync_copy(gtbb_hbm.at[pl.ds(base, CHUNK)], gtbb_v)

    tot = jnp.zeros((16,), jnp.int32)
    prefix = jnp.zeros((16,), jnp.int32)
    for w in range(NW):
        row = hist_v[w]
        tot = tot + row
        prefix = prefix + row * ((w < wid).astype(jnp.int32))
    pc = ((tot + CAP - 1) >> 8) << 8
    po = jnp.cumsum(pc) - pc
    basev = po + prefix

    for g in range(CHUNK // 16):
        cm = cmd_v[pl.ds(g * 16, 16)]
        dest = jnp.zeros((16,), jnp.int32)
        for e in range(NF):
            m = cm == e
            rk = jnp.cumsum(m.astype(jnp.int32)) - 1
            be = jnp.sum(jnp.where(lane == e, basev, 0))
            dest = jnp.where(m, be + rk, dest)
            cnt = plsc.all_reduce_population_count(m)
            basev = basev + jnp.where(lane == e, cnt, 0)
        pos_v[0, pl.ds(g * 16, 16)] = dest

    pltpu.sync_copy(gtbb_v, xgtbb_hbm.at[pos_v.at[0]])
    pltpu.sync_copy(pos_v, pos_hbm.at[wid])

    @pl.when(wid == 0)
    def _():
        blo = jnp.zeros((16,), jnp.int32)
        k1 = lane * CAP
        for e in range(NF):
            po_e = jnp.sum(jnp.where(lane == e, po, 0))
            pc_e = jnp.sum(jnp.where(lane == e, pc, 0))
            blo = jnp.where((k1 >= po_e) & (k1 < po_e + pc_e), e, blo)
        blk_v[pl.ds(0, 16)] = blo
        blk_v[pl.ds(16, 16)] = jnp.zeros((16,), jnp.int32)
        pltpu.sync_copy(blk_v, blk_hbm)


def _s1b_call(cmd, hist, gtbb):
    k = pl.kernel(
        _s1b_body,
        out_type=[jax.ShapeDtypeStruct((NW, 1, CHUNK), jnp.int32),
                  jax.ShapeDtypeStruct((NP, KPAD), jnp.float32),
                  jax.ShapeDtypeStruct((NW,), jnp.int32)],
        mesh=_mesh(),
        scratch_types=[pltpu.VMEM((CHUNK,), jnp.int32),
                       pltpu.VMEM((NW, 16), jnp.int32),
                       pltpu.VMEM((1, CHUNK), jnp.int32),
                       pltpu.VMEM((CHUNK, KPAD), jnp.float32),
                       pltpu.VMEM((NW,), jnp.int32)],
        compiler_params=_sc_params(),
    )
    return k(cmd, hist, gtbb)


def _s1c_body(g_hbm, pos_hbm, xg_hbm, pos_v, g_v):
    wid = _wid()
    pltpu.sync_copy(pos_hbm.at[wid], pos_v)
    pltpu.sync_copy(g_hbm.at[pl.ds(wid * CHUNK, CHUNK)], g_v)
    pltpu.sync_copy(g_v, xg_hbm.at[pos_v.at[0]])


def _s1c_call(g_flat, pos):
    k = pl.kernel(
        _s1c_body,
        out_type=jax.ShapeDtypeStruct((NP, H), jnp.float32),
        mesh=_mesh(),
        scratch_types=[pltpu.VMEM((1, CHUNK), jnp.int32),
                       pltpu.VMEM((CHUNK, H), jnp.float32)],
        compiler_params=_sc_params(),
    )
    return k(g_flat, pos)


def _s2_body(yx_hbm, pos_hbm, mid_hbm, pos_v, y_v):
    wid = _wid()
    pltpu.sync_copy(pos_hbm.at[wid], pos_v)
    pltpu.sync_copy(yx_hbm.at[pos_v.at[0]], y_v)
    pltpu.sync_copy(y_v, mid_hbm.at[wid])


def _s2_call(yx, pos):
    k = pl.kernel(
        _s2_body,
        out_type=jax.ShapeDtypeStruct((NW, CHUNK, 256), jnp.float32),
        mesh=_mesh(),
        scratch_types=[pltpu.VMEM((1, CHUNK), jnp.int32),
                       pltpu.VMEM((CHUNK, 256), jnp.float32)],
        compiler_params=_sc_params(),
    )
    return k(yx, pos)


def _pre_body(x_ref, w1_ref, b1_ref, w2_ref, b2_ref, w3_ref, b3_ref,
              wih_ref, bih_ref, gx_ref):
    x = x_ref[...]
    a = _leaky(_mm(x, w1_ref[...]) + b1_ref[...])
    a = _leaky(_mm(a, w2_ref[...]) + b2_ref[...])
    inp = _mm(a, w3_ref[...]) + b3_ref[...]
    gx_ref[...] = _mm(inp, wih_ref[...]) + bih_ref[...]


def _pre_call(x, w1, b1, w2, b2, w3, b3, wih_t, bih):
    mblk = 256
    full = lambda s: pl.BlockSpec(s, lambda i: (0, 0))
    return pl.pallas_call(
        _pre_body,
        grid=(N // mblk,),
        in_specs=[
            pl.BlockSpec((mblk, KPAD), lambda i: (i, 0)),
            full((KPAD, H)), full((1, H)),
            full((H, H)), full((1, H)),
            full((H, H)), full((1, H)),
            full((H, 3 * H)), full((1, 3 * H)),
        ],
        out_specs=pl.BlockSpec((mblk, 3 * H), lambda i: (i, 0)),
        out_shape=jax.ShapeDtypeStruct((N, 3 * H), jnp.float32),
    )(x, w1, b1, w2, b2, w3, b3, wih_t, bih)


def _gru_body(gx_ref, code_ref, whh_ref, bhh_ref, wcc_ref,
              out_ref, cc_ref, h_ref):
    t = pl.program_id(0)

    @pl.when(t == 0)
    def _():
        h_ref[...] = code_ref[...]
        cc_ref[...] = _mm(code_ref[...], wcc_ref[...])

    h = h_ref[...]
    gh = _mm(h, whh_ref[...]) + bhh_ref[...]
    gx = gx_ref[0]
    r = jax.nn.sigmoid(gx[:, 0:H] + gh[:, 0:H])
    z = jax.nn.sigmoid(gx[:, H:2 * H] + gh[:, H:2 * H])
    n = jnp.tanh(gx[:, 2 * H:] + r * gh[:, 2 * H:])
    hn = (1.0 - z) * n + z * h
    h_ref[...] = hn
    out_ref[0] = hn


def _gru_call(gx, code, whh_t, bhh, wcc):
    return pl.pallas_call(
        _gru_body,
        grid=(T,),
        in_specs=[
            pl.BlockSpec((1, B, 3 * H), lambda t: (t, 0, 0)),
            pl.BlockSpec((B, H), lambda t: (0, 0)),
            pl.BlockSpec((H, 3 * H), lambda t: (0, 0)),
            pl.BlockSpec((1, 3 * H), lambda t: (0, 0)),
            pl.BlockSpec((H, 768), lambda t: (0, 0)),
        ],
        out_specs=[
            pl.BlockSpec((1, B, H), lambda t: (t, 0, 0)),
            pl.BlockSpec((B, 768), lambda t: (0, 0)),
        ],
        out_shape=[
            jax.ShapeDtypeStruct((T, B, H), jnp.float32),
            jax.ShapeDtypeStruct((B, 768), jnp.float32),
        ],
        scratch_shapes=[pltpu.VMEM((B, H), jnp.float32)],
        compiler_params=pltpu.CompilerParams(
            dimension_semantics=("arbitrary",)),
    )(gx, code, whh_t, bhh, wcc)


def _grp_body(blk_ref, xg_ref, xgt_ref,
              w1g_ref, w1t_ref, b1_ref,
              w2d_ref, b2d_ref, w2f_ref, b2f_ref, w2b_ref, b2b_ref,
              w3d_ref, w3f_ref, w3b_ref, b3_ref, p_ref, out_ref):
    m = xg_ref.shape[0]
    term = _mm(xgt_ref[...], w1t_ref[0])
    z = jnp.zeros((m, 256), jnp.float32)
    h1 = _leaky(_mm(xg_ref[...], w1g_ref[0])
                + jnp.concatenate([z, term, z], axis=1) + b1_ref[0])
    h2d = _leaky(_mm(h1[:, 0:256], w2d_ref[0]) + b2d_ref[0])
    h2f = _leaky(_mm(h1[:, 256:512], w2f_ref[0]) + b2f_ref[0])
    h2b = _leaky(_mm(h1[:, 512:768], w2b_ref[0]) + b2b_ref[0])
    y = (_mm(h2d, w3d_ref[0]) + _mm(h2f, w3f_ref[0]) + _mm(h2b, w3b_ref[0])
         + b3_ref[0])
    out_ref[...] = _mm(y, p_ref[0])


def _grp_call(blk, xg, xgt, ew):
    ex = lambda s: pl.BlockSpec((1,) + s, lambda k, b: (b[k], 0, 0))
    grid_spec = pltpu.PrefetchScalarGridSpec(
        num_scalar_prefetch=1,
        grid=(NBLK,),
        in_specs=[
            pl.BlockSpec((CAP, H), lambda k, b: (k, 0)),
            pl.BlockSpec((CAP, KPAD), lambda k, b: (k, 0)),
            ex((H, 768)), ex((KPAD, 256)), ex((1, 768)),
            ex((256, 128)), ex((1, 128)),
            ex((256, 128)), ex((1, 128)),
            ex((256, 128)), ex((1, 128)),
            ex((128, 32)), ex((128, 32)), ex((128, 32)), ex((1, 32)),
            ex((32, 256)),
        ],
        out_specs=pl.BlockSpec((CAP, 256), lambda k, b: (k, 0)),
    )
    return pl.pallas_call(
        _grp_body,
        grid_spec=grid_spec,
        out_shape=jax.ShapeDtypeStruct((NP, 256), jnp.float32),
    )(blk, xg, xgt, *ew)


def _tail_body(g_ref, cc_ref, fu_ref, ch_ref, nc_ref,
               func_ref, child_ref, ncod_ref):
    g = g_ref[...]
    reps = g_ref.shape[0] // B
    ccboth = jnp.concatenate([cc_ref[...]] * reps, axis=0)

    fw1, fb1, fw2, fb2, fw3, fb3 = (fu_ref[i][...] for i in range(6))
    f = _leaky(_mm(g, fw1) + fb1)
    f = _leaky(_mm(f, fw2) + fb2)
    func_ref[...] = _mm(f, fw3) + fb3

    cwg, cb1, cw2, cb2, cw3, cb3 = (ch_ref[i][...] for i in range(6))
    c = _leaky(_mm(g, cwg) + ccboth[:, :256] + cb1)
    c = _leaky(_mm(c, cw2) + cb2)
    child_ref[...] = _mm(c, cw3) + cb3

    nwg, nb1, nw2, nb2, nw3, nb3 = (nc_ref[i][...] for i in range(6))
    nn = _leaky(_mm(g, nwg) + ccboth[:, 256:] + nb1)
    nn = _leaky(_mm(nn, nw2) + nb2)
    res = _mm(nn, nw3) + nb3
    ncod_ref[...] = jnp.swapaxes(res.reshape(-1, B, 4 * H), 0, 1)


def _tail_call(g, cc, fu, ch, nc):
    mblk = 256
    full = lambda a: pl.BlockSpec(a.shape, lambda i: tuple(0 for _ in a.shape))
    return pl.pallas_call(
        _tail_body,
        grid=(N // mblk,),
        in_specs=[
            pl.BlockSpec((mblk, H), lambda i: (i, 0)),
            pl.BlockSpec((B, 768), lambda i: (0, 0)),
            [full(a) for a in fu],
            [full(a) for a in ch],
            [full(a) for a in nc],
        ],
        out_specs=[
            pl.BlockSpec((mblk, NF), lambda i: (i, 0)),
            pl.BlockSpec((mblk, 4), lambda i: (i, 0)),
            pl.BlockSpec((B, mblk // B, 4 * H), lambda i: (0, i, 0)),
        ],
        out_shape=[
            jax.ShapeDtypeStruct((N, NF), jnp.float32),
            jax.ShapeDtypeStruct((N, 4), jnp.float32),
            jax.ShapeDtypeStruct((B, T, 4 * H), jnp.float32),
        ],
    )(g, cc, fu, ch, nc)


def _row(b):
    return b.reshape(1, -1)


def _expert_weights(p):
    disc, fnet, bnet = p["disc"], p["fnet"], p["bnet"]
    hw = H // 2
    z = lambda *s: jnp.zeros(s, jnp.float32)
    w1g, w1t, b1 = [], [], []
    w2d, w2f, w2b = [], [], []
    b2d, b2f, b2b = [], [], []
    w3d, w3f, w3b, b3 = [], [], [], []
    for i in range(NF):
        fW1 = fnet[i][0]["W"]
        w1g.append(jnp.concatenate(
            [disc[i][0]["W"], fW1[:H], bnet[i][0]["W"]], axis=1))
        w1t.append(jnp.concatenate([
            z(NF + 32 * i, hw), fW1[H:H + 24], z(TL - 32 * i - 32, hw),
            fW1[H + 24:], z(KPAD - TL - 3, hw)], axis=0))
        b1.append(jnp.concatenate(
            [disc[i][0]["b"], fnet[i][0]["b"], bnet[i][0]["b"]]).reshape(1, -1))
        w2d.append(disc[i][1]["W"])
        w2f.append(fnet[i][1]["W"])
        w2b.append(bnet[i][1]["W"])
        b2d.append(_row(disc[i][1]["b"]))
        b2f.append(_row(fnet[i][1]["b"]))
        b2b.append(_row(bnet[i][1]["b"]))
        w3d.append(jnp.concatenate([disc[i][2]["W"], z(128, 8)], axis=1))
        w3f.append(jnp.concatenate(
            [z(128, 24), fnet[i][2]["W"], z(128, 2)], axis=1))
        w3b.append(jnp.concatenate([z(128, 30), bnet[i][2]["W"]], axis=1))
        b3.append(jnp.concatenate(
            [disc[i][2]["b"], fnet[i][2]["b"], bnet[i][2]["b"]]).reshape(1, -1))
    pm = np.zeros((NF, 32, 256), np.float32)
    for e in range(NF):
        pm[e, :, 32 * e:32 * e + 32] = np.eye(32, dtype=np.float32)
    st = lambda xs: jnp.stack(xs)
    return [st(w1g), st(w1t), st(b1),
            st(w2d), st(b2d), st(w2f), st(b2f), st(w2b), st(b2b),
            st(w3d), st(w3f), st(w3b), st(b3), jnp.asarray(pm)]


def kernel(inp_seq, code, _bb_dims, _hier_ind, gt_seq, params):
    b, t, tl = inp_seq.shape

    x_t = jnp.swapaxes(inp_seq, 0, 1)
    hier_oh = jax.nn.one_hot(_hier_ind, MAX_DEPTH, dtype=jnp.float32)
    feat = jnp.concatenate(
        [_bb_dims, hier_oh, jnp.zeros((B, KPAD - TL - 7), jnp.float32)],
        axis=1)
    x_cat = jnp.concatenate(
        [x_t, jnp.broadcast_to(feat[None], (T, B, KPAD - TL))], axis=2)
    x_flat = x_cat.reshape(N, KPAD)

    gt_flat = jnp.swapaxes(gt_seq, 0, 1).reshape(N, TL)
    gt8 = gt_flat[:, :NF]
    bb_flat = jnp.broadcast_to(_bb_dims[None], (T, B, 3)).reshape(N, 3)
    gtbb = jnp.concatenate(
        [gt_flat, bb_flat, jnp.zeros((N, KPAD - TL - 3), jnp.float32)],
        axis=1)

    p = params
    inp_net = p["inp_net"]
    w1 = jnp.concatenate(
        [inp_net[0]["W"], jnp.zeros((KPAD - TL - 7, H), jnp.float32)], axis=0)

    cmd, hist = _s1a_call(gt8)
    pos, xgtbb, blk = _s1b_call(cmd, hist, gtbb)

    gru = p["gru"]
    gx = _pre_call(x_flat, w1, _row(inp_net[0]["b"]),
                   inp_net[1]["W"], _row(inp_net[1]["b"]),
                   inp_net[2]["W"], _row(inp_net[2]["b"]),
                   gru["W_ih"].T, _row(gru["b_ih"]))
    wcc = jnp.concatenate(
        [p["child_net"][0]["W"][H:], p["next_code_net"][0]["W"][H:]], axis=1)
    gru_out, cc = _gru_call(gx.reshape(T, B, 3 * H), code, gru["W_hh"].T,
                            _row(gru["b_hh"]), wcc)
    g_flat = gru_out.reshape(N, H)

    xg = _s1c_call(g_flat, pos)
    ew = _expert_weights(p)
    yx = _grp_call(blk, xg, xgtbb, ew)
    mid = _s2_call(yx, pos)

    fu = [p["func_net"][0]["W"], _row(p["func_net"][0]["b"]),
          p["func_net"][1]["W"], _row(p["func_net"][1]["b"]),
          p["func_net"][2]["W"], _row(p["func_net"][2]["b"])]
    ch = [p["child_net"][0]["W"][:H],
          _row(p["child_net"][0]["b"]),
          p["child_net"][1]["W"], _row(p["child_net"][1]["b"]),
          p["child_net"][2]["W"], _row(p["child_net"][2]["b"])]
    nc = [p["next_code_net"][0]["W"][:H],
          _row(p["next_code_net"][0]["b"]),
          p["next_code_net"][1]["W"], _row(p["next_code_net"][1]["b"]),
          p["next_code_net"][2]["W"], _row(p["next_code_net"][2]["b"])]
    func8, child_f, ncod_f = _tail_call(g_flat, cc, fu, ch, nc)

    out_f = jnp.concatenate([func8, mid.reshape(N, 256)], axis=1)
    out = jnp.swapaxes(out_f.reshape(T, B, TL), 0, 1)
    child = jnp.swapaxes(child_f.reshape(T, B, 4), 0, 1)
    ncod = ncod_f.reshape(b, t, 4, H)
    return (out, ncod, child)


# speedup vs baseline: 1.2293x; 1.0209x over previous
import dataclasses
import functools

import jax
import jax.numpy as jnp
import numpy as np
from jax import lax
from jax.experimental import pallas as pl
from jax.experimental.pallas import tpu as pltpu
from jax.experimental.pallas import tpu_sc as plsc

MAX_DEPTH = 4
NF = 8
B, T, H, TL = 32, 64, 512, 264
N = B * T
KPAD = 384
NP = 4096
CAP = 256
NBLK = NP // CAP
NW = 32
CHUNK = N // NW

@functools.cache
def _mesh():
    return plsc.VectorSubcoreMesh(core_axis_name="c", subcore_axis_name="s")


def _sc_params():
    cp = pltpu.CompilerParams()
    if "needs_layout_passes" in pltpu.CompilerParams.__dataclass_fields__:
        cp = dataclasses.replace(cp, needs_layout_passes=False)
    return cp


def _leaky(x):
    return jnp.where(x >= 0, x, 0.2 * x)


def _mm(a, b):
    return a @ b


def _wid():
    return lax.axis_index("s") * 2 + lax.axis_index("c")


def _s1a_body(gt8_hbm, cmd_hbm, hist_hbm, gt8_v, cmd_v, hist_v):
    wid = _wid()
    base = wid * CHUNK
    pltpu.sync_copy(gt8_hbm.at[pl.ds(base, CHUNK)], gt8_v)
    lane = lax.iota(jnp.int32, 16)
    hist = jnp.zeros((16,), jnp.int32)
    for g in range(CHUNK // 16):
        rows = lane + g * 16
        best = plsc.load_gather(gt8_v, [rows, jnp.zeros((16,), jnp.int32)])
        bi = jnp.zeros((16,), jnp.int32)
        for j in range(1, NF):
            colj = plsc.load_gather(
                gt8_v, [rows, jnp.full((16,), j, jnp.int32)])
            m = colj > best
            bi = jnp.where(m, j, bi)
            best = jnp.where(m, colj, best)
        cmd_v[pl.ds(g * 16, 16)] = bi
        for e in range(NF):
            cnt = plsc.all_reduce_population_count(bi == e)
            hist = hist + jnp.where(lane == e, cnt, 0)
    hist_v[...] = hist
    pltpu.sync_copy(cmd_v, cmd_hbm.at[pl.ds(base, CHUNK)])
    pltpu.sync_copy(hist_v, hist_hbm.at[wid])


def _s1a_call(gt8):
    k = pl.kernel(
        _s1a_body,
        out_type=[jax.ShapeDtypeStruct((N,), jnp.int32),
                  jax.ShapeDtypeStruct((NW, 16), jnp.int32)],
        mesh=_mesh(),
        scratch_types=[pltpu.VMEM((CHUNK, NF), jnp.float32),
                       pltpu.VMEM((CHUNK,), jnp.int32),
                       pltpu.VMEM((16,), jnp.int32)],
        compiler_params=_sc_params(),
    )
    return k(gt8)


def _s1b_body(cmd_hbm, hist_hbm, gtbb_hbm,
              pos_hbm, xgtbb_hbm, blk_hbm,
              cmd_v, hist_v, pos_v, gtbb_v, blk_v):
    wid = _wid()
    base = wid * CHUNK
    lane = lax.iota(jnp.int32, 16)
    pltpu.sync_copy(cmd_hbm.at[pl.ds(base, CHUNK)], cmd_v)
    pltpu.sync_copy(hist_hbm, hist_v)
    pltpu.sync_copy(gtbb_hbm.at[pl.ds(base, CHUNK)], gtbb_v)

    tot = jnp.zeros((16,), jnp.int32)
    prefix = jnp.zeros((16,), jnp.int32)
    for w in range(NW):
        row = hist_v[w]
        tot = tot + row
        prefix = prefix + row * ((w < wid).astype(jnp.int32))
    pc = ((tot + CAP - 1) >> 8) << 8
    po = jnp.cumsum(pc) - pc
    basev = po + prefix

    for g in range(CHUNK // 16):
        cm = cmd_v[pl.ds(g * 16, 16)]
        dest = jnp.zeros((16,), jnp.int32)
        for e in range(NF):
            m = cm == e
            rk = jnp.cumsum(m.astype(jnp.int32)) - 1
            be = jnp.sum(jnp.where(lane == e, basev, 0))
            dest = jnp.where(m, be + rk, dest)
            cnt = plsc.all_reduce_population_count(m)
            basev = basev + jnp.where(lane == e, cnt, 0)
        pos_v[0, pl.ds(g * 16, 16)] = dest

    pltpu.sync_copy(gtbb_v, xgtbb_hbm.at[pos_v.at[0]])
    pltpu.sync_copy(pos_v, pos_hbm.at[wid])

    @pl.when(wid == 0)
    def _():
        blo = jnp.zeros((16,), jnp.int32)
        k1 = lane * CAP
        for e in range(NF):
            po_e = jnp.sum(jnp.where(lane == e, po, 0))
            pc_e = jnp.sum(jnp.where(lane == e, pc, 0))
            blo = jnp.where((k1 >= po_e) & (k1 < po_e + pc_e), e, blo)
        blk_v[pl.ds(0, 16)] = blo
        blk_v[pl.ds(16, 16)] = jnp.zeros((16,), jnp.int32)
        pltpu.sync_copy(blk_v, blk_hbm)


def _s1b_call(cmd, hist, gtbb):
    k = pl.kernel(
        _s1b_body,
        out_type=[jax.ShapeDtypeStruct((NW, 1, CHUNK), jnp.int32),
                  jax.ShapeDtypeStruct((NP, KPAD), jnp.float32),
                  jax.ShapeDtypeStruct((NW,), jnp.int32)],
        mesh=_mesh(),
        scratch_types=[pltpu.VMEM((CHUNK,), jnp.int32),
                       pltpu.VMEM((NW, 16), jnp.int32),
                       pltpu.VMEM((1, CHUNK), jnp.int32),
                       pltpu.VMEM((CHUNK, KPAD), jnp.float32),
                       pltpu.VMEM((NW,), jnp.int32)],
        compiler_params=_sc_params(),
    )
    return k(cmd, hist, gtbb)


def _s1c_body(g_hbm, pos_hbm, xg_hbm, pos_v, g_v):
    wid = _wid()
    pltpu.sync_copy(pos_hbm.at[wid], pos_v)
    pltpu.sync_copy(g_hbm.at[pl.ds(wid * CHUNK, CHUNK)], g_v)
    pltpu.sync_copy(g_v, xg_hbm.at[pos_v.at[0]])


def _s1c_call(g_flat, pos):
    k = pl.kernel(
        _s1c_body,
        out_type=jax.ShapeDtypeStruct((NP, H), jnp.float32),
        mesh=_mesh(),
        scratch_types=[pltpu.VMEM((1, CHUNK), jnp.int32),
                       pltpu.VMEM((CHUNK, H), jnp.float32)],
        compiler_params=_sc_params(),
    )
    return k(g_flat, pos)


def _s2_body(yx_hbm, pos_hbm, mid_hbm, pos_v, idx_v, y_v):
    wid = _wid()
    lane = lax.iota(jnp.int32, 16)
    pltpu.sync_copy(pos_hbm.at[wid], pos_v)
    pltpu.sync_copy(yx_hbm.at[pos_v.at[0]], y_v)
    for g in range(CHUNK // 16):
        rl = lane + g * 16
        idx_v[0, pl.ds(g * 16, 16)] = (rl & 31) * T + 2 * wid + (rl >> 5)
    pltpu.sync_copy(y_v, mid_hbm.at[idx_v.at[0]])


def _s2_call(yx, pos):
    k = pl.kernel(
        _s2_body,
        out_type=jax.ShapeDtypeStruct((N, 256), jnp.float32),
        mesh=_mesh(),
        scratch_types=[pltpu.VMEM((1, CHUNK), jnp.int32),
                       pltpu.VMEM((1, CHUNK), jnp.int32),
                       pltpu.VMEM((CHUNK, 256), jnp.float32)],
        compiler_params=_sc_params(),
    )
    return k(yx, pos)


def _pre_body(x_ref, feat_ref, w1_ref, b1_ref, w2_ref, b2_ref, w3_ref,
              b3_ref, wih_ref, bih_ref, gx_ref):
    reps = x_ref.shape[1]
    xr = jnp.swapaxes(x_ref[...], 0, 1).reshape(B * reps, TL)
    x = jnp.concatenate(
        [xr, jnp.concatenate([feat_ref[...]] * reps, axis=0)], axis=1)
    a = _leaky(_mm(x, w1_ref[...]) + b1_ref[...])
    a = _leaky(_mm(a, w2_ref[...]) + b2_ref[...])
    inp = _mm(a, w3_ref[...]) + b3_ref[...]
    gx_ref[...] = _mm(inp, wih_ref[...]) + bih_ref[...]


def _pre_call(x, feat, w1, b1, w2, b2, w3, b3, wih_t, bih):
    mblk = 256
    full = lambda s: pl.BlockSpec(s, lambda i: (0, 0))
    return pl.pallas_call(
        _pre_body,
        grid=(N // mblk,),
        in_specs=[
            pl.BlockSpec((B, mblk // B, TL), lambda i: (0, i, 0)),
            full((B, KPAD - TL)),
            full((KPAD, H)), full((1, H)),
            full((H, H)), full((1, H)),
            full((H, H)), full((1, H)),
            full((H, 3 * H)), full((1, 3 * H)),
        ],
        out_specs=pl.BlockSpec((mblk, 3 * H), lambda i: (i, 0)),
        out_shape=jax.ShapeDtypeStruct((N, 3 * H), jnp.float32),
    )(x, feat, w1, b1, w2, b2, w3, b3, wih_t, bih)


def _gru_body(gx_ref, code_ref, whh_ref, bhh_ref, wcc_ref,
              out_ref, cc_ref, h_ref):
    t = pl.program_id(0)

    @pl.when(t == 0)
    def _():
        h_ref[...] = code_ref[...]
        cc_ref[...] = _mm(code_ref[...], wcc_ref[...])

    h = h_ref[...]
    gh = _mm(h, whh_ref[...]) + bhh_ref[...]
    gx = gx_ref[0]
    r = jax.nn.sigmoid(gx[:, 0:H] + gh[:, 0:H])
    z = jax.nn.sigmoid(gx[:, H:2 * H] + gh[:, H:2 * H])
    n = jnp.tanh(gx[:, 2 * H:] + r * gh[:, 2 * H:])
    hn = (1.0 - z) * n + z * h
    h_ref[...] = hn
    out_ref[0] = hn


def _gru_call(gx, code, whh_t, bhh, wcc):
    return pl.pallas_call(
        _gru_body,
        grid=(T,),
        in_specs=[
            pl.BlockSpec((1, B, 3 * H), lambda t: (t, 0, 0)),
            pl.BlockSpec((B, H), lambda t: (0, 0)),
            pl.BlockSpec((H, 3 * H), lambda t: (0, 0)),
            pl.BlockSpec((1, 3 * H), lambda t: (0, 0)),
            pl.BlockSpec((H, 768), lambda t: (0, 0)),
        ],
        out_specs=[
            pl.BlockSpec((1, B, H), lambda t: (t, 0, 0)),
            pl.BlockSpec((B, 768), lambda t: (0, 0)),
        ],
        out_shape=[
            jax.ShapeDtypeStruct((T, B, H), jnp.float32),
            jax.ShapeDtypeStruct((B, 768), jnp.float32),
        ],
        scratch_shapes=[pltpu.VMEM((B, H), jnp.float32)],
        compiler_params=pltpu.CompilerParams(
            dimension_semantics=("arbitrary",)),
    )(gx, code, whh_t, bhh, wcc)


def _grp_body(blk_ref, xg_ref, xgt_ref,
              w1g_ref, w1t_ref, b1_ref,
              w2d_ref, b2d_ref, w2f_ref, b2f_ref, w2b_ref, b2b_ref,
              w3d_ref, w3f_ref, w3b_ref, b3_ref, p_ref, out_ref):
    m = xg_ref.shape[0]
    term = _mm(xgt_ref[...], w1t_ref[0])
    z = jnp.zeros((m, 256), jnp.float32)
    h1 = _leaky(_mm(xg_ref[...], w1g_ref[0])
                + jnp.concatenate([z, term, z], axis=1) + b1_ref[0])
    h2d = _leaky(_mm(h1[:, 0:256], w2d_ref[0]) + b2d_ref[0])
    h2f = _leaky(_mm(h1[:, 256:512], w2f_ref[0]) + b2f_ref[0])
    h2b = _leaky(_mm(h1[:, 512:768], w2b_ref[0]) + b2b_ref[0])
    y = (_mm(h2d, w3d_ref[0]) + _mm(h2f, w3f_ref[0]) + _mm(h2b, w3b_ref[0])
         + b3_ref[0])
    out_ref[...] = _mm(y, p_ref[0])


def _grp_call(blk, xg, xgt, ew):
    ex = lambda s: pl.BlockSpec((1,) + s, lambda k, b: (b[k], 0, 0))
    grid_spec = pltpu.PrefetchScalarGridSpec(
        num_scalar_prefetch=1,
        grid=(NBLK,),
        in_specs=[
            pl.BlockSpec((CAP, H), lambda k, b: (k, 0)),
            pl.BlockSpec((CAP, KPAD), lambda k, b: (k, 0)),
            ex((H, 768)), ex((KPAD, 256)), ex((1, 768)),
            ex((256, 128)), ex((1, 128)),
            ex((256, 128)), ex((1, 128)),
            ex((256, 128)), ex((1, 128)),
            ex((128, 32)), ex((128, 32)), ex((128, 32)), ex((1, 32)),
            ex((32, 256)),
        ],
        out_specs=pl.BlockSpec((CAP, 256), lambda k, b: (k, 0)),
    )
    return pl.pallas_call(
        _grp_body,
        grid_spec=grid_spec,
        out_shape=jax.ShapeDtypeStruct((NP, 256), jnp.float32),
    )(blk, xg, xgt, *ew)


def _tail_body(g_ref, cc_ref, fu_ref, ch_ref, nc_ref,
               func_ref, child_ref, ncod_ref):
    g = g_ref[...]
    reps = g_ref.shape[0] // B
    ccboth = jnp.concatenate([cc_ref[...]] * reps, axis=0)

    fw1, fb1, fw2, fb2, fw3, fb3 = (fu_ref[i][...] for i in range(6))
    f = _leaky(_mm(g, fw1) + fb1)
    f = _leaky(_mm(f, fw2) + fb2)
    fres = _mm(f, fw3) + fb3
    func_ref[...] = jnp.swapaxes(fres.reshape(-1, B, NF), 0, 1)

    cwg, cb1, cw2, cb2, cw3, cb3 = (ch_ref[i][...] for i in range(6))
    c = _leaky(_mm(g, cwg) + ccboth[:, :256] + cb1)
    c = _leaky(_mm(c, cw2) + cb2)
    cres = _mm(c, cw3) + cb3
    child_ref[...] = jnp.swapaxes(cres.reshape(-1, B, 4), 0, 1)

    nwg, nb1, nw2, nb2, nw3, nb3 = (nc_ref[i][...] for i in range(6))
    nn = _leaky(_mm(g, nwg) + ccboth[:, 256:] + nb1)
    nn = _leaky(_mm(nn, nw2) + nb2)
    res = _mm(nn, nw3) + nb3
    ncod_ref[...] = jnp.swapaxes(res.reshape(-1, B, 4 * H), 0, 1)


def _tail_call(g, cc, fu, ch, nc):
    mblk = 256
    full = lambda a: pl.BlockSpec(a.shape, lambda i: tuple(0 for _ in a.shape))
    return pl.pallas_call(
        _tail_body,
        grid=(N // mblk,),
        in_specs=[
            pl.BlockSpec((mblk, H), lambda i: (i, 0)),
            pl.BlockSpec((B, 768), lambda i: (0, 0)),
            [full(a) for a in fu],
            [full(a) for a in ch],
            [full(a) for a in nc],
        ],
        out_specs=[
            pl.BlockSpec((B, mblk // B, NF), lambda i: (0, i, 0)),
            pl.BlockSpec((B, mblk // B, 4), lambda i: (0, i, 0)),
            pl.BlockSpec((B, mblk // B, 4 * H), lambda i: (0, i, 0)),
        ],
        out_shape=[
            jax.ShapeDtypeStruct((B, T, NF), jnp.float32),
            jax.ShapeDtypeStruct((B, T, 4), jnp.float32),
            jax.ShapeDtypeStruct((B, T, 4 * H), jnp.float32),
        ],
    )(g, cc, fu, ch, nc)


def _row(b):
    return b.reshape(1, -1)


def _expert_weights(p):
    disc, fnet, bnet = p["disc"], p["fnet"], p["bnet"]
    hw = H // 2
    z = lambda *s: jnp.zeros(s, jnp.float32)
    w1g, w1t, b1 = [], [], []
    w2d, w2f, w2b = [], [], []
    b2d, b2f, b2b = [], [], []
    w3d, w3f, w3b, b3 = [], [], [], []
    for i in range(NF):
        fW1 = fnet[i][0]["W"]
        w1g.append(jnp.concatenate(
            [disc[i][0]["W"], fW1[:H], bnet[i][0]["W"]], axis=1))
        w1t.append(jnp.concatenate([
            z(NF + 32 * i, hw), fW1[H:H + 24], z(TL - 32 * i - 32, hw),
            fW1[H + 24:], z(KPAD - TL - 3, hw)], axis=0))
        b1.append(jnp.concatenate(
            [disc[i][0]["b"], fnet[i][0]["b"], bnet[i][0]["b"]]).reshape(1, -1))
        w2d.append(disc[i][1]["W"])
        w2f.append(fnet[i][1]["W"])
        w2b.append(bnet[i][1]["W"])
        b2d.append(_row(disc[i][1]["b"]))
        b2f.append(_row(fnet[i][1]["b"]))
        b2b.append(_row(bnet[i][1]["b"]))
        w3d.append(jnp.concatenate([disc[i][2]["W"], z(128, 8)], axis=1))
        w3f.append(jnp.concatenate(
            [z(128, 24), fnet[i][2]["W"], z(128, 2)], axis=1))
        w3b.append(jnp.concatenate([z(128, 30), bnet[i][2]["W"]], axis=1))
        b3.append(jnp.concatenate(
            [disc[i][2]["b"], fnet[i][2]["b"], bnet[i][2]["b"]]).reshape(1, -1))
    pm = np.zeros((NF, 32, 256), np.float32)
    for e in range(NF):
        pm[e, :, 32 * e:32 * e + 32] = np.eye(32, dtype=np.float32)
    st = lambda xs: jnp.stack(xs)
    return [st(w1g), st(w1t), st(b1),
            st(w2d), st(b2d), st(w2f), st(b2f), st(w2b), st(b2b),
            st(w3d), st(w3f), st(w3b), st(b3), jnp.asarray(pm)]


def kernel(inp_seq, code, _bb_dims, _hier_ind, gt_seq, params):
    b, t, tl = inp_seq.shape

    hier_oh = jax.nn.one_hot(_hier_ind, MAX_DEPTH, dtype=jnp.float32)
    feat = jnp.concatenate(
        [_bb_dims, hier_oh, jnp.zeros((B, KPAD - TL - 7), jnp.float32)],
        axis=1)

    gt_flat = jnp.swapaxes(gt_seq, 0, 1).reshape(N, TL)
    gt8 = gt_flat[:, :NF]
    bb_flat = jnp.broadcast_to(_bb_dims[None], (T, B, 3)).reshape(N, 3)
    gtbb = jnp.concatenate(
        [gt_flat, bb_flat, jnp.zeros((N, KPAD - TL - 3), jnp.float32)],
        axis=1)

    p = params
    inp_net = p["inp_net"]
    w1 = jnp.concatenate(
        [inp_net[0]["W"], jnp.zeros((KPAD - TL - 7, H), jnp.float32)], axis=0)

    cmd, hist = _s1a_call(gt8)
    pos, xgtbb, blk = _s1b_call(cmd, hist, gtbb)

    gru = p["gru"]
    gx = _pre_call(inp_seq, feat, w1, _row(inp_net[0]["b"]),
                   inp_net[1]["W"], _row(inp_net[1]["b"]),
                   inp_net[2]["W"], _row(inp_net[2]["b"]),
                   gru["W_ih"].T, _row(gru["b_ih"]))
    wcc = jnp.concatenate(
        [p["child_net"][0]["W"][H:], p["next_code_net"][0]["W"][H:]], axis=1)
    gru_out, cc = _gru_call(gx.reshape(T, B, 3 * H), code, gru["W_hh"].T,
                            _row(gru["b_hh"]), wcc)
    g_flat = gru_out.reshape(N, H)

    xg = _s1c_call(g_flat, pos)
    ew = _expert_weights(p)
    yx = _grp_call(blk, xg, xgtbb, ew)
    mid = _s2_call(yx, pos)

    fu = [p["func_net"][0]["W"], _row(p["func_net"][0]["b"]),
          p["func_net"][1]["W"], _row(p["func_net"][1]["b"]),
          p["func_net"][2]["W"], _row(p["func_net"][2]["b"])]
    ch = [p["child_net"][0]["W"][:H],
          _row(p["child_net"][0]["b"]),
          p["child_net"][1]["W"], _row(p["child_net"][1]["b"]),
          p["child_net"][2]["W"], _row(p["child_net"][2]["b"])]
    nc = [p["next_code_net"][0]["W"][:H],
          _row(p["next_code_net"][0]["b"]),
          p["next_code_net"][1]["W"], _row(p["next_code_net"][1]["b"]),
          p["next_code_net"][2]["W"], _row(p["next_code_net"][2]["b"])]
    func8, child_f, ncod_f = _tail_call(g_flat, cc, fu, ch, nc)

    out = jnp.concatenate(
        [func8.reshape(N, NF), mid], axis=1).reshape(b, t, tl)
    child = child_f
    ncod = ncod_f.reshape(b, t, 4, H)
    return (out, ncod, child)


# speedup vs baseline: 1.2460x; 1.0136x over previous
import dataclasses
import functools

import jax
import jax.numpy as jnp
import numpy as np
from jax import lax
from jax.experimental import pallas as pl
from jax.experimental.pallas import tpu as pltpu
from jax.experimental.pallas import tpu_sc as plsc

MAX_DEPTH = 4
NF = 8
B, T, H, TL = 32, 64, 512, 264
N = B * T
KPAD = 384
NP = 4096
CAP = 256
NBLK = NP // CAP
NW = 32
CHUNK = N // NW

@functools.cache
def _mesh():
    return plsc.VectorSubcoreMesh(core_axis_name="c", subcore_axis_name="s")


def _sc_params():
    cp = pltpu.CompilerParams()
    if "needs_layout_passes" in pltpu.CompilerParams.__dataclass_fields__:
        cp = dataclasses.replace(cp, needs_layout_passes=False)
    return cp


def _leaky(x):
    return jnp.where(x >= 0, x, 0.2 * x)


def _mm(a, b):
    return a @ b


def _wid():
    return lax.axis_index("s") * 2 + lax.axis_index("c")


def _s1a_body(gt8_hbm, cmd_hbm, hist_hbm, gt8_v, cmd_v, hist_v):
    wid = _wid()
    base = wid * CHUNK
    pltpu.sync_copy(gt8_hbm.at[pl.ds(base, CHUNK)], gt8_v)
    lane = lax.iota(jnp.int32, 16)
    hist = jnp.zeros((16,), jnp.int32)
    for g in range(CHUNK // 16):
        rows = lane + g * 16
        best = plsc.load_gather(gt8_v, [rows, jnp.zeros((16,), jnp.int32)])
        bi = jnp.zeros((16,), jnp.int32)
        for j in range(1, NF):
            colj = plsc.load_gather(
                gt8_v, [rows, jnp.full((16,), j, jnp.int32)])
            m = colj > best
            bi = jnp.where(m, j, bi)
            best = jnp.where(m, colj, best)
        cmd_v[pl.ds(g * 16, 16)] = bi
        for e in range(NF):
            cnt = plsc.all_reduce_population_count(bi == e)
            hist = hist + jnp.where(lane == e, cnt, 0)
    hist_v[...] = hist
    pltpu.sync_copy(cmd_v, cmd_hbm.at[pl.ds(base, CHUNK)])
    pltpu.sync_copy(hist_v, hist_hbm.at[wid])


def _s1a_call(gt8):
    k = pl.kernel(
        _s1a_body,
        out_type=[jax.ShapeDtypeStruct((N,), jnp.int32),
                  jax.ShapeDtypeStruct((NW, 16), jnp.int32)],
        mesh=_mesh(),
        scratch_types=[pltpu.VMEM((CHUNK, NF), jnp.float32),
                       pltpu.VMEM((CHUNK,), jnp.int32),
                       pltpu.VMEM((16,), jnp.int32)],
        compiler_params=_sc_params(),
    )
    return k(gt8)


def _s1b_body(cmd_hbm, hist_hbm, gtbb_hbm,
              pos_hbm, xgtbb_hbm, blk_hbm,
              cmd_v, hist_v, pos_v, gtbb_v, blk_v):
    wid = _wid()
    base = wid * CHUNK
    lane = lax.iota(jnp.int32, 16)
    pltpu.sync_copy(cmd_hbm.at[pl.ds(base, CHUNK)], cmd_v)
    pltpu.sync_copy(hist_hbm, hist_v)
    pltpu.sync_copy(gtbb_hbm.at[pl.ds(base, CHUNK)], gtbb_v)

    tot = jnp.zeros((16,), jnp.int32)
    prefix = jnp.zeros((16,), jnp.int32)
    for w in range(NW):
        row = hist_v[w]
        tot = tot + row
        prefix = prefix + row * ((w < wid).astype(jnp.int32))
    pc = ((tot + CAP - 1) >> 8) << 8
    po = jnp.cumsum(pc) - pc
    basev = po + prefix

    for g in range(CHUNK // 16):
        cm = cmd_v[pl.ds(g * 16, 16)]
        dest = jnp.zeros((16,), jnp.int32)
        for e in range(NF):
            m = cm == e
            rk = jnp.cumsum(m.astype(jnp.int32)) - 1
            be = jnp.sum(jnp.where(lane == e, basev, 0))
            dest = jnp.where(m, be + rk, dest)
            cnt = plsc.all_reduce_population_count(m)
            basev = basev + jnp.where(lane == e, cnt, 0)
        pos_v[0, pl.ds(g * 16, 16)] = dest

    pltpu.sync_copy(gtbb_v, xgtbb_hbm.at[pos_v.at[0]])
    pltpu.sync_copy(pos_v, pos_hbm.at[wid])

    @pl.when(wid == 0)
    def _():
        blo = jnp.zeros((16,), jnp.int32)
        k1 = lane * CAP
        for e in range(NF):
            po_e = jnp.sum(jnp.where(lane == e, po, 0))
            pc_e = jnp.sum(jnp.where(lane == e, pc, 0))
            blo = jnp.where((k1 >= po_e) & (k1 < po_e + pc_e), e, blo)
        blk_v[pl.ds(0, 16)] = blo
        blk_v[pl.ds(16, 16)] = jnp.zeros((16,), jnp.int32)
        pltpu.sync_copy(blk_v, blk_hbm)


def _s1b_call(cmd, hist, gtbb):
    k = pl.kernel(
        _s1b_body,
        out_type=[jax.ShapeDtypeStruct((NW, 1, CHUNK), jnp.int32),
                  jax.ShapeDtypeStruct((NP, KPAD), jnp.float32),
                  jax.ShapeDtypeStruct((NW,), jnp.int32)],
        mesh=_mesh(),
        scratch_types=[pltpu.VMEM((CHUNK,), jnp.int32),
                       pltpu.VMEM((NW, 16), jnp.int32),
                       pltpu.VMEM((1, CHUNK), jnp.int32),
                       pltpu.VMEM((CHUNK, KPAD), jnp.float32),
                       pltpu.VMEM((NW,), jnp.int32)],
        compiler_params=_sc_params(),
    )
    return k(cmd, hist, gtbb)


def _s1c_body(g_hbm, pos_hbm, xg_hbm, pos_v, g_v):
    wid = _wid()
    pltpu.sync_copy(pos_hbm.at[wid], pos_v)
    pltpu.sync_copy(g_hbm.at[pl.ds(wid * CHUNK, CHUNK)], g_v)
    pltpu.sync_copy(g_v, xg_hbm.at[pos_v.at[0]])


def _s1c_call(g_flat, pos):
    k = pl.kernel(
        _s1c_body,
        out_type=jax.ShapeDtypeStruct((NP, H), jnp.float32),
        mesh=_mesh(),
        scratch_types=[pltpu.VMEM((1, CHUNK), jnp.int32),
                       pltpu.VMEM((CHUNK, H), jnp.float32)],
        compiler_params=_sc_params(),
    )
    return k(g_flat, pos)


def _s2_body(yx_hbm, pos_hbm, mid_hbm, pos_v, idx_v, y_v):
    wid = _wid()
    lane = lax.iota(jnp.int32, 16)
    pltpu.sync_copy(pos_hbm.at[wid], pos_v)
    pltpu.sync_copy(yx_hbm.at[pos_v.at[0]], y_v)
    for g in range(CHUNK // 16):
        rl = lane + g * 16
        idx_v[0, pl.ds(g * 16, 16)] = (rl & 31) * T + 2 * wid + (rl >> 5)
    pltpu.sync_copy(y_v, mid_hbm.at[idx_v.at[0]])


def _s2_call(yx, pos):
    k = pl.kernel(
        _s2_body,
        out_type=jax.ShapeDtypeStruct((N, 256), jnp.float32),
        mesh=_mesh(),
        scratch_types=[pltpu.VMEM((1, CHUNK), jnp.int32),
                       pltpu.VMEM((1, CHUNK), jnp.int32),
                       pltpu.VMEM((CHUNK, 256), jnp.float32)],
        compiler_params=_sc_params(),
    )
    return k(yx, pos)


def _pre_body(x_ref, feat_ref, w1_ref, b1_ref, w2_ref, b2_ref, w3_ref,
              b3_ref, wih_ref, bih_ref, gx_ref):
    reps = x_ref.shape[1]
    xr = jnp.swapaxes(x_ref[...], 0, 1).reshape(B * reps, TL)
    x = jnp.concatenate(
        [xr, jnp.concatenate([feat_ref[...]] * reps, axis=0)], axis=1)
    a = _leaky(_mm(x, w1_ref[...]) + b1_ref[...])
    a = _leaky(_mm(a, w2_ref[...]) + b2_ref[...])
    inp = _mm(a, w3_ref[...]) + b3_ref[...]
    gx_ref[...] = _mm(inp, wih_ref[...]) + bih_ref[...]


def _pre_call(x, feat, w1, b1, w2, b2, w3, b3, wih_t, bih):
    mblk = 256
    full = lambda s: pl.BlockSpec(s, lambda i: (0, 0))
    return pl.pallas_call(
        _pre_body,
        grid=(N // mblk,),
        in_specs=[
            pl.BlockSpec((B, mblk // B, TL), lambda i: (0, i, 0)),
            full((B, KPAD - TL)),
            full((KPAD, H)), full((1, H)),
            full((H, H)), full((1, H)),
            full((H, H)), full((1, H)),
            full((H, 3 * H)), full((1, 3 * H)),
        ],
        out_specs=pl.BlockSpec((mblk, 3 * H), lambda i: (i, 0)),
        out_shape=jax.ShapeDtypeStruct((N, 3 * H), jnp.float32),
    )(x, feat, w1, b1, w2, b2, w3, b3, wih_t, bih)


def _gru_body(gx_ref, code_ref, whh_ref, bhh_ref, out_ref, h_ref):
    t = pl.program_id(0)

    @pl.when(t == 0)
    def _():
        h_ref[...] = code_ref[...]

    h = h_ref[...]
    gh = _mm(h, whh_ref[...]) + bhh_ref[...]
    gx = gx_ref[0]
    r = jax.nn.sigmoid(gx[:, 0:H] + gh[:, 0:H])
    z = jax.nn.sigmoid(gx[:, H:2 * H] + gh[:, H:2 * H])
    n = jnp.tanh(gx[:, 2 * H:] + r * gh[:, 2 * H:])
    hn = (1.0 - z) * n + z * h
    h_ref[...] = hn
    out_ref[0] = hn


def _gru_call(gx, code, whh_t, bhh):
    return pl.pallas_call(
        _gru_body,
        grid=(T,),
        in_specs=[
            pl.BlockSpec((1, B, 3 * H), lambda t: (t, 0, 0)),
            pl.BlockSpec((B, H), lambda t: (0, 0)),
            pl.BlockSpec((H, 3 * H), lambda t: (0, 0)),
            pl.BlockSpec((1, 3 * H), lambda t: (0, 0)),
        ],
        out_specs=pl.BlockSpec((1, B, H), lambda t: (t, 0, 0)),
        out_shape=jax.ShapeDtypeStruct((T, B, H), jnp.float32),
        scratch_shapes=[pltpu.VMEM((B, H), jnp.float32)],
        compiler_params=pltpu.CompilerParams(
            dimension_semantics=("arbitrary",)),
    )(gx, code, whh_t, bhh)


def _grp_body(blk_ref, xg_ref, xgt_ref,
              w1g_ref, w1t_ref, b1_ref,
              w2d_ref, b2d_ref, w2f_ref, b2f_ref, w2b_ref, b2b_ref,
              w3d_ref, w3f_ref, w3b_ref, b3_ref, p_ref, out_ref):
    m = xg_ref.shape[0]
    term = _mm(xgt_ref[...], w1t_ref[0])
    z = jnp.zeros((m, 256), jnp.float32)
    h1 = _leaky(_mm(xg_ref[...], w1g_ref[0])
                + jnp.concatenate([z, term, z], axis=1) + b1_ref[0])
    h2d = _leaky(_mm(h1[:, 0:256], w2d_ref[0]) + b2d_ref[0])
    h2f = _leaky(_mm(h1[:, 256:512], w2f_ref[0]) + b2f_ref[0])
    h2b = _leaky(_mm(h1[:, 512:768], w2b_ref[0]) + b2b_ref[0])
    y = (_mm(h2d, w3d_ref[0]) + _mm(h2f, w3f_ref[0]) + _mm(h2b, w3b_ref[0])
         + b3_ref[0])
    out_ref[...] = _mm(y, p_ref[0])


def _grp_call(blk, xg, xgt, ew):
    ex = lambda s: pl.BlockSpec((1,) + s, lambda k, b: (b[k], 0, 0))
    grid_spec = pltpu.PrefetchScalarGridSpec(
        num_scalar_prefetch=1,
        grid=(NBLK,),
        in_specs=[
            pl.BlockSpec((CAP, H), lambda k, b: (k, 0)),
            pl.BlockSpec((CAP, KPAD), lambda k, b: (k, 0)),
            ex((H, 768)), ex((KPAD, 256)), ex((1, 768)),
            ex((256, 128)), ex((1, 128)),
            ex((256, 128)), ex((1, 128)),
            ex((256, 128)), ex((1, 128)),
            ex((128, 32)), ex((128, 32)), ex((128, 32)), ex((1, 32)),
            ex((32, 256)),
        ],
        out_specs=pl.BlockSpec((CAP, 256), lambda k, b: (k, 0)),
    )
    return pl.pallas_call(
        _grp_body,
        grid_spec=grid_spec,
        out_shape=jax.ShapeDtypeStruct((NP, 256), jnp.float32),
    )(blk, xg, xgt, *ew)


def _tail_body(g_ref, code_ref, fu_ref, ch_ref, nc_ref,
               func_ref, child_ref, ncod_ref):
    g = g_ref[...]
    reps = g_ref.shape[0] // B
    code = code_ref[...]

    fw1, fb1, fw2, fb2, fw3, fb3 = (fu_ref[i][...] for i in range(6))
    f = _leaky(_mm(g, fw1) + fb1)
    f = _leaky(_mm(f, fw2) + fb2)
    fres = _mm(f, fw3) + fb3
    func_ref[...] = jnp.swapaxes(fres.reshape(-1, B, NF), 0, 1)

    cwg, cwc, cb1, cw2, cb2, cw3, cb3 = (ch_ref[i][...] for i in range(7))
    cc = jnp.concatenate([_mm(code, cwc)] * reps, axis=0)
    c = _leaky(_mm(g, cwg) + cc + cb1)
    c = _leaky(_mm(c, cw2) + cb2)
    cres = _mm(c, cw3) + cb3
    child_ref[...] = jnp.swapaxes(cres.reshape(-1, B, 4), 0, 1)

    nwg, nwc, nb1, nw2, nb2, nw3, nb3 = (nc_ref[i][...] for i in range(7))
    ncc = jnp.concatenate([_mm(code, nwc)] * reps, axis=0)
    nn = _leaky(_mm(g, nwg) + ncc + nb1)
    nn = _leaky(_mm(nn, nw2) + nb2)
    res = _mm(nn, nw3) + nb3
    ncod_ref[...] = jnp.swapaxes(res.reshape(-1, B, 4 * H), 0, 1)


def _tail_call(g, code, fu, ch, nc):
    mblk = 256
    full = lambda a: pl.BlockSpec(a.shape, lambda i: tuple(0 for _ in a.shape))
    return pl.pallas_call(
        _tail_body,
        grid=(N // mblk,),
        in_specs=[
            pl.BlockSpec((mblk, H), lambda i: (i, 0)),
            pl.BlockSpec((B, H), lambda i: (0, 0)),
            [full(a) for a in fu],
            [full(a) for a in ch],
            [full(a) for a in nc],
        ],
        out_specs=[
            pl.BlockSpec((B, mblk // B, NF), lambda i: (0, i, 0)),
            pl.BlockSpec((B, mblk // B, 4), lambda i: (0, i, 0)),
            pl.BlockSpec((B, mblk // B, 4 * H), lambda i: (0, i, 0)),
        ],
        out_shape=[
            jax.ShapeDtypeStruct((B, T, NF), jnp.float32),
            jax.ShapeDtypeStruct((B, T, 4), jnp.float32),
            jax.ShapeDtypeStruct((B, T, 4 * H), jnp.float32),
        ],
    )(g, code, fu, ch, nc)


def _row(b):
    return b.reshape(1, -1)


def _expert_weights(p):
    disc, fnet, bnet = p["disc"], p["fnet"], p["bnet"]
    hw = H // 2
    z = lambda *s: jnp.zeros(s, jnp.float32)
    w1g, w1t, b1 = [], [], []
    w2d, w2f, w2b = [], [], []
    b2d, b2f, b2b = [], [], []
    w3d, w3f, w3b, b3 = [], [], [], []
    for i in range(NF):
        fW1 = fnet[i][0]["W"]
        w1g.append(jnp.concatenate(
            [disc[i][0]["W"], fW1[:H], bnet[i][0]["W"]], axis=1))
        w1t.append(jnp.concatenate([
            z(NF + 32 * i, hw), fW1[H:H + 24], z(TL - 32 * i - 32, hw),
            fW1[H + 24:], z(KPAD - TL - 3, hw)], axis=0))
        b1.append(jnp.concatenate(
            [disc[i][0]["b"], fnet[i][0]["b"], bnet[i][0]["b"]]).reshape(1, -1))
        w2d.append(disc[i][1]["W"])
        w2f.append(fnet[i][1]["W"])
        w2b.append(bnet[i][1]["W"])
        b2d.append(_row(disc[i][1]["b"]))
        b2f.append(_row(fnet[i][1]["b"]))
        b2b.append(_row(bnet[i][1]["b"]))
        w3d.append(jnp.concatenate([disc[i][2]["W"], z(128, 8)], axis=1))
        w3f.append(jnp.concatenate(
            [z(128, 24), fnet[i][2]["W"], z(128, 2)], axis=1))
        w3b.append(jnp.concatenate([z(128, 30), bnet[i][2]["W"]], axis=1))
        b3.append(jnp.concatenate(
            [disc[i][2]["b"], fnet[i][2]["b"], bnet[i][2]["b"]]).reshape(1, -1))
    pm = np.zeros((NF, 32, 256), np.float32)
    for e in range(NF):
        pm[e, :, 32 * e:32 * e + 32] = np.eye(32, dtype=np.float32)
    st = lambda xs: jnp.stack(xs)
    return [st(w1g), st(w1t), st(b1),
            st(w2d), st(b2d), st(w2f), st(b2f), st(w2b), st(b2b),
            st(w3d), st(w3f), st(w3b), st(b3), jnp.asarray(pm)]


def kernel(inp_seq, code, _bb_dims, _hier_ind, gt_seq, params):
    b, t, tl = inp_seq.shape

    hier_oh = jax.nn.one_hot(_hier_ind, MAX_DEPTH, dtype=jnp.float32)
    feat = jnp.concatenate(
        [_bb_dims, hier_oh, jnp.zeros((B, KPAD - TL - 7), jnp.float32)],
        axis=1)

    gt_flat = jnp.swapaxes(gt_seq, 0, 1).reshape(N, TL)
    gt8 = gt_flat[:, :NF]
    bb_flat = jnp.broadcast_to(_bb_dims[None], (T, B, 3)).reshape(N, 3)
    gtbb = jnp.concatenate(
        [gt_flat, bb_flat, jnp.zeros((N, KPAD - TL - 3), jnp.float32)],
        axis=1)

    p = params
    inp_net = p["inp_net"]
    w1 = jnp.concatenate(
        [inp_net[0]["W"], jnp.zeros((KPAD - TL - 7, H), jnp.float32)], axis=0)

    cmd, hist = _s1a_call(gt8)
    pos, xgtbb, blk = _s1b_call(cmd, hist, gtbb)

    gru = p["gru"]
    gx = _pre_call(inp_seq, feat, w1, _row(inp_net[0]["b"]),
                   inp_net[1]["W"], _row(inp_net[1]["b"]),
                   inp_net[2]["W"], _row(inp_net[2]["b"]),
                   gru["W_ih"].T, _row(gru["b_ih"]))
    gru_out = _gru_call(gx.reshape(T, B, 3 * H), code, gru["W_hh"].T,
                        _row(gru["b_hh"]))
    g_flat = gru_out.reshape(N, H)

    xg = _s1c_call(g_flat, pos)
    ew = _expert_weights(p)
    yx = _grp_call(blk, xg, xgtbb, ew)
    mid = _s2_call(yx, pos)

    fu = [p["func_net"][0]["W"], _row(p["func_net"][0]["b"]),
          p["func_net"][1]["W"], _row(p["func_net"][1]["b"]),
          p["func_net"][2]["W"], _row(p["func_net"][2]["b"])]
    ch = [p["child_net"][0]["W"][:H], p["child_net"][0]["W"][H:],
          _row(p["child_net"][0]["b"]),
          p["child_net"][1]["W"], _row(p["child_net"][1]["b"]),
          p["child_net"][2]["W"], _row(p["child_net"][2]["b"])]
    nc = [p["next_code_net"][0]["W"][:H], p["next_code_net"][0]["W"][H:],
          _row(p["next_code_net"][0]["b"]),
          p["next_code_net"][1]["W"], _row(p["next_code_net"][1]["b"]),
          p["next_code_net"][2]["W"], _row(p["next_code_net"][2]["b"])]
    func8, child_f, ncod_f = _tail_call(g_flat, code, fu, ch, nc)

    out = jnp.concatenate(
        [func8.reshape(N, NF), mid], axis=1).reshape(b, t, tl)
    child = child_f
    ncod = ncod_f.reshape(b, t, 4, H)
    return (out, ncod, child)


# speedup vs baseline: 1.2767x; 1.0246x over previous
import dataclasses
import functools

import jax
import jax.numpy as jnp
import numpy as np
from jax import lax
from jax.experimental import pallas as pl
from jax.experimental.pallas import tpu as pltpu
from jax.experimental.pallas import tpu_sc as plsc

MAX_DEPTH = 4
NF = 8
B, T, H, TL = 32, 64, 512, 264
N = B * T
KPAD = 384
NP = 4096
CAP = 256
NBLK = NP // CAP
NW = 32
CHUNK = N // NW

@functools.cache
def _mesh():
    return plsc.VectorSubcoreMesh(core_axis_name="c", subcore_axis_name="s")


def _sc_params():
    cp = pltpu.CompilerParams()
    if "needs_layout_passes" in pltpu.CompilerParams.__dataclass_fields__:
        cp = dataclasses.replace(cp, needs_layout_passes=False)
    return cp


def _leaky(x):
    return jnp.where(x >= 0, x, 0.2 * x)


def _mm(a, b):
    return a @ b


def _wid():
    return lax.axis_index("s") * 2 + lax.axis_index("c")


def _s1a_body(gt8_hbm, cmd_hbm, hist_hbm, gt8_v, cmd_v, hist_v):
    wid = _wid()
    base = wid * CHUNK
    pltpu.sync_copy(gt8_hbm.at[pl.ds(base, CHUNK)], gt8_v)
    lane = lax.iota(jnp.int32, 16)
    hist = jnp.zeros((16,), jnp.int32)
    for g in range(CHUNK // 16):
        rows = lane + g * 16
        best = plsc.load_gather(gt8_v, [rows, jnp.zeros((16,), jnp.int32)])
        bi = jnp.zeros((16,), jnp.int32)
        for j in range(1, NF):
            colj = plsc.load_gather(
                gt8_v, [rows, jnp.full((16,), j, jnp.int32)])
            m = colj > best
            bi = jnp.where(m, j, bi)
            best = jnp.where(m, colj, best)
        cmd_v[pl.ds(g * 16, 16)] = bi
        for e in range(NF):
            cnt = plsc.all_reduce_population_count(bi == e)
            hist = hist + jnp.where(lane == e, cnt, 0)
    hist_v[...] = hist
    pltpu.sync_copy(cmd_v, cmd_hbm.at[pl.ds(base, CHUNK)])
    pltpu.sync_copy(hist_v, hist_hbm.at[wid])


def _s1a_call(gt8):
    k = pl.kernel(
        _s1a_body,
        out_type=[jax.ShapeDtypeStruct((N,), jnp.int32),
                  jax.ShapeDtypeStruct((NW, 16), jnp.int32)],
        mesh=_mesh(),
        scratch_types=[pltpu.VMEM((CHUNK, NF), jnp.float32),
                       pltpu.VMEM((CHUNK,), jnp.int32),
                       pltpu.VMEM((16,), jnp.int32)],
        compiler_params=_sc_params(),
    )
    return k(gt8)


def _s1b_body(cmd_hbm, hist_hbm, gtbb_hbm,
              pos_hbm, xgtbb_hbm, blk_hbm,
              cmd_v, hist_v, pos_v, gtbb_v, blk_v):
    wid = _wid()
    base = wid * CHUNK
    lane = lax.iota(jnp.int32, 16)
    pltpu.sync_copy(cmd_hbm.at[pl.ds(base, CHUNK)], cmd_v)
    pltpu.sync_copy(hist_hbm, hist_v)
    pltpu.sync_copy(gtbb_hbm.at[pl.ds(base, CHUNK)], gtbb_v)

    tot = jnp.zeros((16,), jnp.int32)
    prefix = jnp.zeros((16,), jnp.int32)
    for w in range(NW):
        row = hist_v[w]
        tot = tot + row
        prefix = prefix + row * ((w < wid).astype(jnp.int32))
    pc = ((tot + CAP - 1) >> 8) << 8
    po = jnp.cumsum(pc) - pc
    basev = po + prefix

    for g in range(CHUNK // 16):
        cm = cmd_v[pl.ds(g * 16, 16)]
        dest = jnp.zeros((16,), jnp.int32)
        for e in range(NF):
            m = cm == e
            rk = jnp.cumsum(m.astype(jnp.int32)) - 1
            be = jnp.sum(jnp.where(lane == e, basev, 0))
            dest = jnp.where(m, be + rk, dest)
            cnt = plsc.all_reduce_population_count(m)
            basev = basev + jnp.where(lane == e, cnt, 0)
        pos_v[0, pl.ds(g * 16, 16)] = dest

    pltpu.sync_copy(gtbb_v, xgtbb_hbm.at[pos_v.at[0]])
    pltpu.sync_copy(pos_v, pos_hbm.at[wid])

    @pl.when(wid == 0)
    def _():
        blo = jnp.zeros((16,), jnp.int32)
        k1 = lane * CAP
        for e in range(NF):
            po_e = jnp.sum(jnp.where(lane == e, po, 0))
            pc_e = jnp.sum(jnp.where(lane == e, pc, 0))
            blo = jnp.where((k1 >= po_e) & (k1 < po_e + pc_e), e, blo)
        blk_v[pl.ds(0, 16)] = blo
        blk_v[pl.ds(16, 16)] = jnp.zeros((16,), jnp.int32)
        pltpu.sync_copy(blk_v, blk_hbm)


def _s1b_call(cmd, hist, gtbb):
    k = pl.kernel(
        _s1b_body,
        out_type=[jax.ShapeDtypeStruct((NW, 1, CHUNK), jnp.int32),
                  jax.ShapeDtypeStruct((NP, KPAD), jnp.float32),
                  jax.ShapeDtypeStruct((NW,), jnp.int32)],
        mesh=_mesh(),
        scratch_types=[pltpu.VMEM((CHUNK,), jnp.int32),
                       pltpu.VMEM((NW, 16), jnp.int32),
                       pltpu.VMEM((1, CHUNK), jnp.int32),
                       pltpu.VMEM((CHUNK, KPAD), jnp.float32),
                       pltpu.VMEM((NW,), jnp.int32)],
        compiler_params=_sc_params(),
    )
    return k(cmd, hist, gtbb)


def _s1c_body(g_hbm, pos_hbm, xg_hbm, pos_v, g_v):
    wid = _wid()
    pltpu.sync_copy(pos_hbm.at[wid], pos_v)
    pltpu.sync_copy(g_hbm.at[pl.ds(wid * CHUNK, CHUNK)], g_v)
    pltpu.sync_copy(g_v, xg_hbm.at[pos_v.at[0]])


def _s1c_call(g_flat, pos):
    k = pl.kernel(
        _s1c_body,
        out_type=jax.ShapeDtypeStruct((NP, H), jnp.float32),
        mesh=_mesh(),
        scratch_types=[pltpu.VMEM((1, CHUNK), jnp.int32),
                       pltpu.VMEM((CHUNK, H), jnp.float32)],
        compiler_params=_sc_params(),
    )
    return k(g_flat, pos)


def _s2_body(yx_hbm, pos_hbm, mid_hbm, pos_v, idx_v, y_v):
    wid = _wid()
    lane = lax.iota(jnp.int32, 16)
    pltpu.sync_copy(pos_hbm.at[wid], pos_v)
    pltpu.sync_copy(yx_hbm.at[pos_v.at[0]], y_v)
    for g in range(CHUNK // 16):
        rl = lane + g * 16
        idx_v[0, pl.ds(g * 16, 16)] = (rl & 31) * T + 2 * wid + (rl >> 5)
    pltpu.sync_copy(y_v, mid_hbm.at[idx_v.at[0]])


def _s2_call(yx, pos):
    k = pl.kernel(
        _s2_body,
        out_type=jax.ShapeDtypeStruct((N, 256), jnp.float32),
        mesh=_mesh(),
        scratch_types=[pltpu.VMEM((1, CHUNK), jnp.int32),
                       pltpu.VMEM((1, CHUNK), jnp.int32),
                       pltpu.VMEM((CHUNK, 256), jnp.float32)],
        compiler_params=_sc_params(),
    )
    return k(yx, pos)


def _pre_body(x_ref, feat_ref, w1_ref, b1_ref, w2_ref, b2_ref, w3_ref,
              b3_ref, wih_ref, bih_ref, gx_ref):
    reps = x_ref.shape[1]
    xr = jnp.swapaxes(x_ref[...], 0, 1).reshape(B * reps, TL)
    x = jnp.concatenate(
        [xr, jnp.concatenate([feat_ref[...]] * reps, axis=0)], axis=1)
    a = _leaky(_mm(x, w1_ref[...]) + b1_ref[...])
    a = _leaky(_mm(a, w2_ref[...]) + b2_ref[...])
    inp = _mm(a, w3_ref[...]) + b3_ref[...]
    gx_ref[...] = _mm(inp, wih_ref[...]) + bih_ref[...]


def _pre_call(x, feat, w1, b1, w2, b2, w3, b3, wih_t, bih):
    mblk = 512
    full = lambda s: pl.BlockSpec(s, lambda i: (0, 0))
    return pl.pallas_call(
        _pre_body,
        grid=(N // mblk,),
        in_specs=[
            pl.BlockSpec((B, mblk // B, TL), lambda i: (0, i, 0)),
            full((B, KPAD - TL)),
            full((KPAD, H)), full((1, H)),
            full((H, H)), full((1, H)),
            full((H, H)), full((1, H)),
            full((H, 3 * H)), full((1, 3 * H)),
        ],
        out_specs=pl.BlockSpec((mblk, 3 * H), lambda i: (i, 0)),
        out_shape=jax.ShapeDtypeStruct((N, 3 * H), jnp.float32),
    )(x, feat, w1, b1, w2, b2, w3, b3, wih_t, bih)


def _gru_body(gx_ref, code_ref, whh_ref, bhh_ref, out_ref, h_ref):
    t = pl.program_id(0)

    @pl.when(t == 0)
    def _():
        h_ref[...] = code_ref[...]

    h = h_ref[...]
    gh = _mm(h, whh_ref[...]) + bhh_ref[...]
    gx = gx_ref[0]
    r = jax.nn.sigmoid(gx[:, 0:H] + gh[:, 0:H])
    z = jax.nn.sigmoid(gx[:, H:2 * H] + gh[:, H:2 * H])
    n = jnp.tanh(gx[:, 2 * H:] + r * gh[:, 2 * H:])
    hn = (1.0 - z) * n + z * h
    h_ref[...] = hn
    out_ref[0] = hn


def _gru_call(gx, code, whh_t, bhh):
    return pl.pallas_call(
        _gru_body,
        grid=(T,),
        in_specs=[
            pl.BlockSpec((1, B, 3 * H), lambda t: (t, 0, 0)),
            pl.BlockSpec((B, H), lambda t: (0, 0)),
            pl.BlockSpec((H, 3 * H), lambda t: (0, 0)),
            pl.BlockSpec((1, 3 * H), lambda t: (0, 0)),
        ],
        out_specs=pl.BlockSpec((1, B, H), lambda t: (t, 0, 0)),
        out_shape=jax.ShapeDtypeStruct((T, B, H), jnp.float32),
        scratch_shapes=[pltpu.VMEM((B, H), jnp.float32)],
        compiler_params=pltpu.CompilerParams(
            dimension_semantics=("arbitrary",)),
    )(gx, code, whh_t, bhh)


def _grp_body(blk_ref, xg_ref, xgt_ref,
              w1g_ref, w1t_ref, b1_ref,
              w2d_ref, b2d_ref, w2f_ref, b2f_ref, w2b_ref, b2b_ref,
              w3d_ref, w3f_ref, w3b_ref, b3_ref, p_ref, out_ref):
    m = xg_ref.shape[0]
    term = _mm(xgt_ref[...], w1t_ref[0])
    z = jnp.zeros((m, 256), jnp.float32)
    h1 = _leaky(_mm(xg_ref[...], w1g_ref[0])
                + jnp.concatenate([z, term, z], axis=1) + b1_ref[0])
    h2d = _leaky(_mm(h1[:, 0:256], w2d_ref[0]) + b2d_ref[0])
    h2f = _leaky(_mm(h1[:, 256:512], w2f_ref[0]) + b2f_ref[0])
    h2b = _leaky(_mm(h1[:, 512:768], w2b_ref[0]) + b2b_ref[0])
    y = (_mm(h2d, w3d_ref[0]) + _mm(h2f, w3f_ref[0]) + _mm(h2b, w3b_ref[0])
         + b3_ref[0])
    out_ref[...] = _mm(y, p_ref[0])


def _grp_call(blk, xg, xgt, ew):
    ex = lambda s: pl.BlockSpec((1,) + s, lambda k, b: (b[k], 0, 0))
    grid_spec = pltpu.PrefetchScalarGridSpec(
        num_scalar_prefetch=1,
        grid=(NBLK,),
        in_specs=[
            pl.BlockSpec((CAP, H), lambda k, b: (k, 0)),
            pl.BlockSpec((CAP, KPAD), lambda k, b: (k, 0)),
            ex((H, 768)), ex((KPAD, 256)), ex((1, 768)),
            ex((256, 128)), ex((1, 128)),
            ex((256, 128)), ex((1, 128)),
            ex((256, 128)), ex((1, 128)),
            ex((128, 32)), ex((128, 32)), ex((128, 32)), ex((1, 32)),
            ex((32, 256)),
        ],
        out_specs=pl.BlockSpec((CAP, 256), lambda k, b: (k, 0)),
    )
    return pl.pallas_call(
        _grp_body,
        grid_spec=grid_spec,
        out_shape=jax.ShapeDtypeStruct((NP, 256), jnp.float32),
    )(blk, xg, xgt, *ew)


def _tail_body(g_ref, code_ref, fu_ref, ch_ref, nc_ref,
               func_ref, child_ref, ncod_ref):
    g = g_ref[...]
    reps = g_ref.shape[0] // B
    code = code_ref[...]

    fw1, fb1, fw2, fb2, fw3, fb3 = (fu_ref[i][...] for i in range(6))
    f = _leaky(_mm(g, fw1) + fb1)
    f = _leaky(_mm(f, fw2) + fb2)
    fres = _mm(f, fw3) + fb3
    func_ref[...] = jnp.swapaxes(fres.reshape(-1, B, NF), 0, 1)

    cwg, cwc, cb1, cw2, cb2, cw3, cb3 = (ch_ref[i][...] for i in range(7))
    cc = jnp.concatenate([_mm(code, cwc)] * reps, axis=0)
    c = _leaky(_mm(g, cwg) + cc + cb1)
    c = _leaky(_mm(c, cw2) + cb2)
    cres = _mm(c, cw3) + cb3
    child_ref[...] = jnp.swapaxes(cres.reshape(-1, B, 4), 0, 1)

    nwg, nwc, nb1, nw2, nb2, nw3, nb3 = (nc_ref[i][...] for i in range(7))
    ncc = jnp.concatenate([_mm(code, nwc)] * reps, axis=0)
    nn = _leaky(_mm(g, nwg) + ncc + nb1)
    nn = _leaky(_mm(nn, nw2) + nb2)
    res = _mm(nn, nw3) + nb3
    ncod_ref[...] = jnp.swapaxes(res.reshape(-1, B, 4 * H), 0, 1)


def _tail_call(g, code, fu, ch, nc):
    mblk = 512
    full = lambda a: pl.BlockSpec(a.shape, lambda i: tuple(0 for _ in a.shape))
    return pl.pallas_call(
        _tail_body,
        grid=(N // mblk,),
        in_specs=[
            pl.BlockSpec((mblk, H), lambda i: (i, 0)),
            pl.BlockSpec((B, H), lambda i: (0, 0)),
            [full(a) for a in fu],
            [full(a) for a in ch],
            [full(a) for a in nc],
        ],
        out_specs=[
            pl.BlockSpec((B, mblk // B, NF), lambda i: (0, i, 0)),
            pl.BlockSpec((B, mblk // B, 4), lambda i: (0, i, 0)),
            pl.BlockSpec((B, mblk // B, 4 * H), lambda i: (0, i, 0)),
        ],
        out_shape=[
            jax.ShapeDtypeStruct((B, T, NF), jnp.float32),
            jax.ShapeDtypeStruct((B, T, 4), jnp.float32),
            jax.ShapeDtypeStruct((B, T, 4 * H), jnp.float32),
        ],
    )(g, code, fu, ch, nc)


def _row(b):
    return b.reshape(1, -1)


def _expert_weights(p):
    disc, fnet, bnet = p["disc"], p["fnet"], p["bnet"]
    hw = H // 2
    z = lambda *s: jnp.zeros(s, jnp.float32)
    w1g, w1t, b1 = [], [], []
    w2d, w2f, w2b = [], [], []
    b2d, b2f, b2b = [], [], []
    w3d, w3f, w3b, b3 = [], [], [], []
    for i in range(NF):
        fW1 = fnet[i][0]["W"]
        w1g.append(jnp.concatenate(
            [disc[i][0]["W"], fW1[:H], bnet[i][0]["W"]], axis=1))
        w1t.append(jnp.concatenate([
            z(NF + 32 * i, hw), fW1[H:H + 24], z(TL - 32 * i - 32, hw),
            fW1[H + 24:], z(KPAD - TL - 3, hw)], axis=0))
        b1.append(jnp.concatenate(
            [disc[i][0]["b"], fnet[i][0]["b"], bnet[i][0]["b"]]).reshape(1, -1))
        w2d.append(disc[i][1]["W"])
        w2f.append(fnet[i][1]["W"])
        w2b.append(bnet[i][1]["W"])
        b2d.append(_row(disc[i][1]["b"]))
        b2f.append(_row(fnet[i][1]["b"]))
        b2b.append(_row(bnet[i][1]["b"]))
        w3d.append(jnp.concatenate([disc[i][2]["W"], z(128, 8)], axis=1))
        w3f.append(jnp.concatenate(
            [z(128, 24), fnet[i][2]["W"], z(128, 2)], axis=1))
        w3b.append(jnp.concatenate([z(128, 30), bnet[i][2]["W"]], axis=1))
        b3.append(jnp.concatenate(
            [disc[i][2]["b"], fnet[i][2]["b"], bnet[i][2]["b"]]).reshape(1, -1))
    pm = np.zeros((NF, 32, 256), np.float32)
    for e in range(NF):
        pm[e, :, 32 * e:32 * e + 32] = np.eye(32, dtype=np.float32)
    st = lambda xs: jnp.stack(xs)
    return [st(w1g), st(w1t), st(b1),
            st(w2d), st(b2d), st(w2f), st(b2f), st(w2b), st(b2b),
            st(w3d), st(w3f), st(w3b), st(b3), jnp.asarray(pm)]


def kernel(inp_seq, code, _bb_dims, _hier_ind, gt_seq, params):
    b, t, tl = inp_seq.shape

    hier_oh = jax.nn.one_hot(_hier_ind, MAX_DEPTH, dtype=jnp.float32)
    feat = jnp.concatenate(
        [_bb_dims, hier_oh, jnp.zeros((B, KPAD - TL - 7), jnp.float32)],
        axis=1)

    gt_flat = jnp.swapaxes(gt_seq, 0, 1).reshape(N, TL)
    gt8 = gt_flat[:, :NF]
    bb_flat = jnp.broadcast_to(_bb_dims[None], (T, B, 3)).reshape(N, 3)
    gtbb = jnp.concatenate(
        [gt_flat, bb_flat, jnp.zeros((N, KPAD - TL - 3), jnp.float32)],
        axis=1)

    p = params
    inp_net = p["inp_net"]
    w1 = jnp.concatenate(
        [inp_net[0]["W"], jnp.zeros((KPAD - TL - 7, H), jnp.float32)], axis=0)

    cmd, hist = _s1a_call(gt8)
    pos, xgtbb, blk = _s1b_call(cmd, hist, gtbb)

    gru = p["gru"]
    gx = _pre_call(inp_seq, feat, w1, _row(inp_net[0]["b"]),
                   inp_net[1]["W"], _row(inp_net[1]["b"]),
                   inp_net[2]["W"], _row(inp_net[2]["b"]),
                   gru["W_ih"].T, _row(gru["b_ih"]))
    gru_out = _gru_call(gx.reshape(T, B, 3 * H), code, gru["W_hh"].T,
                        _row(gru["b_hh"]))
    g_flat = gru_out.reshape(N, H)

    xg = _s1c_call(g_flat, pos)
    ew = _expert_weights(p)
    yx = _grp_call(blk, xg, xgtbb, ew)
    mid = _s2_call(yx, pos)

    fu = [p["func_net"][0]["W"], _row(p["func_net"][0]["b"]),
          p["func_net"][1]["W"], _row(p["func_net"][1]["b"]),
          p["func_net"][2]["W"], _row(p["func_net"][2]["b"])]
    ch = [p["child_net"][0]["W"][:H], p["child_net"][0]["W"][H:],
          _row(p["child_net"][0]["b"]),
          p["child_net"][1]["W"], _row(p["child_net"][1]["b"]),
          p["child_net"][2]["W"], _row(p["child_net"][2]["b"])]
    nc = [p["next_code_net"][0]["W"][:H], p["next_code_net"][0]["W"][H:],
          _row(p["next_code_net"][0]["b"]),
          p["next_code_net"][1]["W"], _row(p["next_code_net"][1]["b"]),
          p["next_code_net"][2]["W"], _row(p["next_code_net"][2]["b"])]
    func8, child_f, ncod_f = _tail_call(g_flat, code, fu, ch, nc)

    out = jnp.concatenate(
        [func8.reshape(N, NF), mid], axis=1).reshape(b, t, tl)
    child = child_f
    ncod = ncod_f.reshape(b, t, 4, H)
    return (out, ncod, child)
